# Initial kernel scaffold; baseline (speedup 1.0000x reference)
import functools

import jax
import jax.numpy as jnp
from jax import lax
from jax.experimental import pallas as pl
from jax.experimental.pallas import tpu as pltpu
from jax.experimental.pallas import tpu_sc as plsc

NU = 25000
NI = 25000
NN = NU + NI
EMB = 128
NF = 4
DF = EMB // NF
E_RAW = 600000
EH_RAW = E_RAW // 2

NC = 2
NS = 16
CH = 128
EH = 307200
E2 = 2 * EH
EPW = EH // NS
NCHUNK = EPW // CH

NPH = 25088
NP = 2 * NPH
TPN = NPH // NS
ACC = NF * NPH
SLICE = ACC // NS


def _newton_rsqrt(x, iters=3):
    i = lax.bitcast_convert_type(x, jnp.int32)
    y = lax.bitcast_convert_type(jnp.int32(0x5F3759DF) - (i >> 1), jnp.float32)
    for _ in range(iters):
        y = y * (1.5 - 0.5 * x * y * y)
    return y


def _softmax4(abuf, j):
    a = [abuf[f, pl.ds(j * 16, 16)] for f in range(NF)]
    m = jnp.maximum(jnp.maximum(a[0], a[1]), jnp.maximum(a[2], a[3]))
    e = [jnp.exp(v - m) for v in a]
    r = 1.0 / (e[0] + e[1] + e[2] + e[3])
    return e, r


def _mesh():
    return plsc.VectorSubcoreMesh(core_axis_name="c", subcore_axis_name="s")


@functools.cache
def _make_dsum():
    @functools.partial(
        pl.kernel,
        mesh=_mesh(),
        out_type=jax.ShapeDtypeStruct((NF, NP), jnp.float32),
        scratch_types=[
            pltpu.VMEM((ACC,), jnp.float32),
            pltpu.VMEM((NF, CH), jnp.float32),
            pltpu.VMEM((CH,), jnp.int32),
            pltpu.VMEM((SLICE,), jnp.float32),
            pltpu.VMEM((SLICE,), jnp.float32),
            pltpu.VMEM_SHARED((NS, ACC), jnp.float32),
        ],
    )
    def dsum(a_hbm, hl_hbm, d_hbm, acc, abuf, hbuf, tmp, ssum, spart):
        c = lax.axis_index("c")
        s = lax.axis_index("s")
        zero = jnp.zeros((16,), jnp.float32)

        def zbody(i, _):
            acc[pl.ds(i * 16, 16)] = zero
            return 0

        lax.fori_loop(0, ACC // 16, zbody, 0)

        def chunk(i, _):
            gbase = c * EH + s * EPW + i * CH
            pltpu.sync_copy(a_hbm.at[:, pl.ds(gbase, CH)], abuf)
            pltpu.sync_copy(hl_hbm.at[pl.ds(gbase, CH)], hbuf)
            for j in range(CH // 16):
                e, r = _softmax4(abuf, j)
                idx = hbuf[pl.ds(j * 16, 16)]
                for f in range(NF):
                    plsc.addupdate_scatter(acc, [idx + f * NPH], e[f] * r)
            return 0

        lax.fori_loop(0, NCHUNK, chunk, 0)

        pltpu.sync_copy(acc, spart.at[s])
        plsc.subcore_barrier()

        def zsum(i, _):
            ssum[pl.ds(i * 16, 16)] = zero
            return 0

        lax.fori_loop(0, SLICE // 16, zsum, 0)
        for t in range(NS):
            pltpu.sync_copy(spart.at[t, pl.ds(s * SLICE, SLICE)], tmp)

            def addb(i, _):
                ssum[pl.ds(i * 16, 16)] = (
                    ssum[pl.ds(i * 16, 16)] + tmp[pl.ds(i * 16, 16)]
                )
                return 0

            lax.fori_loop(0, SLICE // 16, addb, 0)

        def finb(i, _):
            x = jnp.maximum(ssum[pl.ds(i * 16, 16)], 1e-8)
            ssum[pl.ds(i * 16, 16)] = _newton_rsqrt(x)
            return 0

        lax.fori_loop(0, SLICE // 16, finb, 0)
        f = s // 4
        nbase = (s % 4) * SLICE
        pltpu.sync_copy(ssum, d_hbm.at[f, pl.ds(c * NPH + nbase, SLICE)])

    return dsum


@functools.cache
def _make_message(f: int):
    @functools.partial(
        pl.kernel,
        mesh=_mesh(),
        out_type=jax.ShapeDtypeStruct((NP, DF), jnp.float32),
        scratch_types=[
            pltpu.VMEM((NP,), jnp.float32),
            pltpu.VMEM((NF, CH), jnp.float32),
            pltpu.VMEM((CH,), jnp.int32),
            pltpu.VMEM((CH,), jnp.int32),
            pltpu.VMEM((CH, DF), jnp.float32),
            pltpu.VMEM((CH, DF), jnp.float32),
            pltpu.VMEM_SHARED((NPH, DF), jnp.float32),
        ],
    )
    def message(a_hbm, d_hbm, t_dense_hbm, hl_hbm, tpg_hbm, f_hbm,
                dbuf, abuf, hlbuf, tbufi, trow, vrow, facc):
        c = lax.axis_index("c")
        s = lax.axis_index("s")
        pltpu.sync_copy(d_hbm.at[f], dbuf)

        zero = jnp.zeros((16,), jnp.float32)

        def zvec(i, _):
            vrow[i // 2, pl.ds((i % 2) * 16, 16)] = zero
            return 0

        lax.fori_loop(0, CH * 2, zvec, 0)
        base_row = s * TPN
        for b in range(TPN // CH):
            pltpu.sync_copy(vrow, facc.at[pl.ds(base_row + b * CH, CH)])
        rem = TPN - (TPN // CH) * CH
        pltpu.sync_copy(vrow.at[pl.ds(0, rem)],
                        facc.at[pl.ds(base_row + (TPN // CH) * CH, rem)])
        plsc.subcore_barrier()

        def chunk(i, _):
            gbase = c * EH + s * EPW + i * CH
            pltpu.sync_copy(a_hbm.at[:, pl.ds(gbase, CH)], abuf)
            pltpu.sync_copy(hl_hbm.at[pl.ds(gbase, CH)], hlbuf)
            pltpu.sync_copy(tpg_hbm.at[pl.ds(gbase, CH)], tbufi)
            pltpu.sync_copy(t_dense_hbm.at[pl.ds(gbase, CH), pl.ds(f * DF, DF)],
                            trow)
            for j in range(CH // 16):
                e, r = _softmax4(abuf, j)
                tp = e[f] * r
                hloc = hlbuf[pl.ds(j * 16, 16)]
                tglb = tbufi[pl.ds(j * 16, 16)]
                dh = plsc.load_gather(dbuf, [hloc + c * NPH])
                dt = plsc.load_gather(dbuf, [tglb])
                ew = tp * dh * dt
                for j2 in range(16):
                    row = j * 16 + j2
                    w = ew[j2]
                    vrow[row, pl.ds(0, 16)] = trow[row, pl.ds(0, 16)] * w
                    vrow[row, pl.ds(16, 16)] = trow[row, pl.ds(16, 16)] * w
            pltpu.sync_copy(vrow, facc.at[hlbuf], add=True)
            return 0

        lax.fori_loop(0, NCHUNK, chunk, 0)
        plsc.subcore_barrier()
        pltpu.sync_copy(facc.at[pl.ds(base_row, TPN)],
                        f_hbm.at[pl.ds(c * NPH + base_row, TPN)])

    return message


@functools.cache
def _make_gather4():
    @functools.partial(
        pl.kernel,
        mesh=_mesh(),
        out_type=jax.ShapeDtypeStruct((E2, EMB), jnp.float32),
        scratch_types=[
            pltpu.VMEM((CH,), jnp.int32),
            pltpu.VMEM((CH, DF), jnp.float32),
            pltpu.SemaphoreType.DMA,
        ],
    )
    def gather4(idx_hbm, tab0, tab1, tab2, tab3, out_hbm, ibuf, rbuf, sem):
        c = lax.axis_index("c")
        s = lax.axis_index("s")
        tabs = [tab0, tab1, tab2, tab3]
        for f in range(NF):
            def chunk(i, _, f=f):
                gbase = c * EH + s * EPW + i * CH
                pltpu.sync_copy(idx_hbm.at[pl.ds(gbase, CH)], ibuf)
                pltpu.async_copy(tabs[f].at[ibuf], rbuf, sem).wait()
                pltpu.sync_copy(rbuf,
                                out_hbm.at[pl.ds(gbase, CH), pl.ds(f * DF, DF)])
                return 0

            lax.fori_loop(0, NCHUNK, chunk, 0)

    return gather4


_BE = 1536


def _att_body(a_ref, g_ref, t_ref, o_ref):
    for f in range(NF):
        g = g_ref[:, f * DF:(f + 1) * DF]
        t = t_ref[:, f * DF:(f + 1) * DF]
        gn = g / jnp.maximum(jnp.sqrt(jnp.sum(g * g, axis=1, keepdims=True)),
                             1e-12)
        tn = t / jnp.maximum(jnp.sqrt(jnp.sum(t * t, axis=1, keepdims=True)),
                             1e-12)
        o_ref[f, :] = a_ref[f, :] + jnp.sum(gn * jnp.tanh(tn), axis=1)


@functools.cache
def _make_att():
    return pl.pallas_call(
        _att_body,
        grid=(E2 // _BE,),
        in_specs=[
            pl.BlockSpec((NF, _BE), lambda i: (0, i)),
            pl.BlockSpec((_BE, EMB), lambda i: (i, 0)),
            pl.BlockSpec((_BE, EMB), lambda i: (i, 0)),
        ],
        out_specs=pl.BlockSpec((NF, _BE), lambda i: (0, i)),
        out_shape=jax.ShapeDtypeStruct((NF, E2), jnp.float32),
    )


_BN = 1024


def _mean_body(*refs):
    ins, o_ref = refs[:-1], refs[-1]
    for f in range(NF):
        acc = ins[f][...] + ins[NF + f][...] + ins[2 * NF + f][...]
        o_ref[:, f * DF:(f + 1) * DF] = acc * (1.0 / 3.0)


@functools.cache
def _make_mean():
    return pl.pallas_call(
        _mean_body,
        grid=(NP // _BN,),
        in_specs=[pl.BlockSpec((_BN, DF), lambda i: (i, 0))
                  for _ in range(3 * NF)],
        out_specs=pl.BlockSpec((_BN, EMB), lambda i: (i, 0)),
        out_shape=jax.ShapeDtypeStruct((NP, EMB), jnp.float32),
    )


def kernel(user_emb, item_emb, all_h_list, all_t_list):
    h = all_h_list
    t = all_t_list
    hpg = h + jnp.where(h >= NU, NP - NN, 0).astype(jnp.int32)
    tpg = t + jnp.where(t >= NU, NP - NN, 0).astype(jnp.int32)
    padn = EH - EH_RAW
    hpg2 = jnp.concatenate([
        hpg[:EH_RAW], jnp.full((padn,), NU, jnp.int32),
        hpg[EH_RAW:], jnp.full((padn,), NPH + NU, jnp.int32),
    ])
    tpg2 = jnp.concatenate([
        tpg[:EH_RAW], jnp.zeros((padn,), jnp.int32),
        tpg[EH_RAW:], jnp.zeros((padn,), jnp.int32),
    ])
    hl2 = jnp.concatenate([hpg2[:EH], hpg2[EH:] - NPH])

    zpad = jnp.zeros((NPH - NU, DF), jnp.float32)
    ego = [
        jnp.concatenate([user_emb[:, f * DF:(f + 1) * DF], zpad,
                         item_emb[:, f * DF:(f + 1) * DF], zpad], axis=0)
        for f in range(NF)
    ]

    dsum = _make_dsum()
    gather4 = _make_gather4()
    att = _make_att()
    msg = [_make_message(f) for f in range(NF)]

    a_val = jnp.ones((NF, E2), jnp.float32)
    all_layers = [ego]
    for layer in range(2):
        tails = gather4(tpg2, *ego)
        layer_f = None
        for it in range(2):
            d = dsum(a_val, hl2)
            layer_f = [msg[f](a_val, d, tails, hl2, tpg2) for f in range(NF)]
            last = layer == 1 and it == 1
            if not last:
                heads = gather4(hpg2, *layer_f)
                a_val = att(a_val, heads, tails)
        ego = layer_f
        all_layers.append(ego)

    mean = _make_mean()
    emb = mean(*[tab for lay in all_layers for tab in lay])
    u_g = emb[:NU, :]
    i_g = emb[NPH:NPH + NI, :]
    return (u_g, i_g)


# speedup vs baseline: 8.7041x; 8.7041x over previous
import functools

import jax
import jax.numpy as jnp
from jax import lax
from jax.experimental import pallas as pl
from jax.experimental.pallas import tpu as pltpu
from jax.experimental.pallas import tpu_sc as plsc

NU = 25000
NI = 25000
NN = NU + NI
EMB = 128
NF = 4
DF = EMB // NF
E_RAW = 600000
EH_RAW = E_RAW // 2

NC = 2
NS = 16
CH = 128
EH = 307200
E2 = 2 * EH
EPW = EH // NS
NCHUNK = EPW // CH

NPH = 25088
NP = 2 * NPH
TPN = NPH // NS
ACC = NF * NPH
SLICE = ACC // NS


def _newton_rsqrt(x, iters=3):
    i = lax.bitcast_convert_type(x, jnp.int32)
    y = lax.bitcast_convert_type(jnp.int32(0x5F3759DF) - (i >> 1), jnp.float32)
    for _ in range(iters):
        y = y * (1.5 - 0.5 * x * y * y)
    return y


def _softmax4(abuf, j):
    a = [abuf[f, pl.ds(j * 16, 16)] for f in range(NF)]
    m = jnp.maximum(jnp.maximum(a[0], a[1]), jnp.maximum(a[2], a[3]))
    e = [jnp.exp(v - m) for v in a]
    r = 1.0 / (e[0] + e[1] + e[2] + e[3])
    return e, r


def _mesh():
    return plsc.VectorSubcoreMesh(core_axis_name="c", subcore_axis_name="s")


_SC_PARAMS = pltpu.CompilerParams(needs_layout_passes=False,
                                  use_tc_tiling_on_sc=False)


@functools.cache
def _make_dsum():
    @functools.partial(
        pl.kernel,
        mesh=_mesh(),
        compiler_params=_SC_PARAMS,
        out_type=tuple(jax.ShapeDtypeStruct((NP,), jnp.float32)
                       for _ in range(NF)),
        scratch_types=[
            pltpu.VMEM((NF, CH), jnp.float32),
            pltpu.VMEM((CH,), jnp.int32),
            pltpu.VMEM((CH,), jnp.float32),
            pltpu.VMEM((CH,), jnp.float32),
            pltpu.VMEM((CH,), jnp.float32),
            pltpu.VMEM((CH,), jnp.float32),
            pltpu.VMEM((TPN,), jnp.float32),
            pltpu.VMEM_SHARED((NPH,), jnp.float32),
            pltpu.VMEM_SHARED((NPH,), jnp.float32),
            pltpu.VMEM_SHARED((NPH,), jnp.float32),
            pltpu.VMEM_SHARED((NPH,), jnp.float32),
        ],
    )
    def dsum(a_hbm, hl_hbm, d0, d1, d2, d3, abuf, hbuf, tp0, tp1, tp2, tp3,
             lslice, acc0, acc1, acc2, acc3):
        d_out = [d0, d1, d2, d3]
        tpb = [tp0, tp1, tp2, tp3]
        accsh = [acc0, acc1, acc2, acc3]
        c = lax.axis_index("c")
        s = lax.axis_index("s")
        zero = jnp.zeros((16,), jnp.float32)

        def zbody(i, _):
            lslice[pl.ds(i * 16, 16)] = zero
            return 0

        lax.fori_loop(0, TPN // 16, zbody, 0)
        base_row = s * TPN
        for f in range(NF):
            pltpu.sync_copy(lslice, accsh[f].at[pl.ds(base_row, TPN)])
        plsc.subcore_barrier()

        def chunk(i, _):
            gbase = c * EH + s * EPW + i * CH
            pltpu.sync_copy(a_hbm.at[:, pl.ds(gbase, CH)], abuf)
            pltpu.sync_copy(hl_hbm.at[pl.ds(gbase, CH)], hbuf)
            for j in range(CH // 16):
                e, r = _softmax4(abuf, j)
                for f in range(NF):
                    tpb[f][pl.ds(j * 16, 16)] = e[f] * r
            for f in range(NF):
                pltpu.sync_copy(tpb[f], accsh[f].at[hbuf], add=True)
            return 0

        lax.fori_loop(0, NCHUNK, chunk, 0)
        plsc.subcore_barrier()

        for f in range(NF):
            pltpu.sync_copy(accsh[f].at[pl.ds(base_row, TPN)], lslice)

            def finb(i, _):
                x = jnp.maximum(lslice[pl.ds(i * 16, 16)], 1e-8)
                lslice[pl.ds(i * 16, 16)] = _newton_rsqrt(x)
                return 0

            lax.fori_loop(0, TPN // 16, finb, 0)
            pltpu.sync_copy(lslice,
                            d_out[f].at[pl.ds(c * NPH + base_row, TPN)])

    return dsum


@functools.cache
def _make_message(f: int):
    @functools.partial(
        pl.kernel,
        mesh=_mesh(),
        compiler_params=_SC_PARAMS,
        out_type=jax.ShapeDtypeStruct((NP, DF), jnp.float32),
        scratch_types=[
            pltpu.VMEM((NF, CH), jnp.float32),
            pltpu.VMEM((CH,), jnp.int32),
            pltpu.VMEM((CH,), jnp.int32),
            pltpu.VMEM((CH,), jnp.int32),
            pltpu.VMEM((CH,), jnp.float32),
            pltpu.VMEM((CH,), jnp.float32),
            pltpu.VMEM((CH, DF), jnp.float32),
            pltpu.VMEM((CH, DF), jnp.float32),
            pltpu.VMEM_SHARED((NPH, DF), jnp.float32),
        ],
    )
    def message(a_hbm, d_hbm, t_dense_hbm, hl_hbm, tpg_hbm, f_hbm,
                abuf, hlbuf, hpgbuf, tbufi, dhbuf, dtbuf, trow, vrow, facc):
        c = lax.axis_index("c")
        s = lax.axis_index("s")

        zero = jnp.zeros((16,), jnp.float32)
        for i in range(CH):
            vrow[i, pl.ds(0, 16)] = zero
            vrow[i, pl.ds(16, 16)] = zero
        base_row = s * TPN
        for b in range(TPN // CH):
            pltpu.sync_copy(vrow, facc.at[pl.ds(base_row + b * CH, CH)])
        rem = TPN - (TPN // CH) * CH
        pltpu.sync_copy(vrow.at[pl.ds(0, rem)],
                        facc.at[pl.ds(base_row + (TPN // CH) * CH, rem)])
        plsc.subcore_barrier()

        def chunk(i, _):
            gbase = c * EH + s * EPW + i * CH
            pltpu.sync_copy(a_hbm.at[:, pl.ds(gbase, CH)], abuf)
            pltpu.sync_copy(hl_hbm.at[pl.ds(gbase, CH)], hlbuf)
            pltpu.sync_copy(tpg_hbm.at[pl.ds(gbase, CH)], tbufi)
            pltpu.sync_copy(t_dense_hbm.at[f, pl.ds(gbase, CH), :], trow)
            for j in range(CH // 16):
                hpgbuf[pl.ds(j * 16, 16)] = (
                    hlbuf[pl.ds(j * 16, 16)] + c * NPH
                )
            pltpu.sync_copy(d_hbm.at[hpgbuf], dhbuf)
            pltpu.sync_copy(d_hbm.at[tbufi], dtbuf)
            for j in range(CH // 16):
                e, r = _softmax4(abuf, j)
                tp = e[f] * r
                dh = dhbuf[pl.ds(j * 16, 16)]
                dt = dtbuf[pl.ds(j * 16, 16)]
                ew = tp * dh * dt
                for j2 in range(16):
                    row = j * 16 + j2
                    w = ew[j2]
                    vrow[row, pl.ds(0, 16)] = trow[row, pl.ds(0, 16)] * w
                    vrow[row, pl.ds(16, 16)] = trow[row, pl.ds(16, 16)] * w
            pltpu.sync_copy(vrow, facc.at[hlbuf], add=True)
            return 0

        lax.fori_loop(0, NCHUNK, chunk, 0)
        plsc.subcore_barrier()
        pltpu.sync_copy(facc.at[pl.ds(base_row, TPN)],
                        f_hbm.at[pl.ds(c * NPH + base_row, TPN)])

    return message


@functools.cache
def _make_gather4():
    @functools.partial(
        pl.kernel,
        mesh=_mesh(),
        compiler_params=_SC_PARAMS,
        out_type=jax.ShapeDtypeStruct((NF, E2, DF), jnp.float32),
        scratch_types=[
            pltpu.VMEM((CH,), jnp.int32),
            pltpu.VMEM((CH, DF), jnp.float32),
            pltpu.SemaphoreType.DMA,
        ],
    )
    def gather4(idx_hbm, tab0, tab1, tab2, tab3, out_hbm, ibuf, rbuf, sem):
        c = lax.axis_index("c")
        s = lax.axis_index("s")
        tabs = [tab0, tab1, tab2, tab3]
        for f in range(NF):
            def chunk(i, _, f=f):
                gbase = c * EH + s * EPW + i * CH
                pltpu.sync_copy(idx_hbm.at[pl.ds(gbase, CH)], ibuf)
                pltpu.async_copy(tabs[f].at[ibuf], rbuf, sem).wait()
                pltpu.sync_copy(rbuf, out_hbm.at[f, pl.ds(gbase, CH), :])
                return 0

            lax.fori_loop(0, NCHUNK, chunk, 0)

    return gather4


_BE = 1536


def _att_body(a_ref, g_ref, t_ref, o_ref):
    for f in range(NF):
        g = g_ref[f]
        t = t_ref[f]
        gn = g / jnp.maximum(jnp.sqrt(jnp.sum(g * g, axis=1, keepdims=True)),
                             1e-12)
        tn = t / jnp.maximum(jnp.sqrt(jnp.sum(t * t, axis=1, keepdims=True)),
                             1e-12)
        o_ref[f, :] = a_ref[f, :] + jnp.sum(gn * jnp.tanh(tn), axis=1)


@functools.cache
def _make_att():
    return pl.pallas_call(
        _att_body,
        grid=(E2 // _BE,),
        in_specs=[
            pl.BlockSpec((NF, _BE), lambda i: (0, i)),
            pl.BlockSpec((NF, _BE, DF), lambda i: (0, i, 0)),
            pl.BlockSpec((NF, _BE, DF), lambda i: (0, i, 0)),
        ],
        out_specs=pl.BlockSpec((NF, _BE), lambda i: (0, i)),
        out_shape=jax.ShapeDtypeStruct((NF, E2), jnp.float32),
    )


_BN = 1024


def _mean_body(*refs):
    ins, o_ref = refs[:-1], refs[-1]
    for f in range(NF):
        acc = ins[f][...] + ins[NF + f][...] + ins[2 * NF + f][...]
        o_ref[:, f * DF:(f + 1) * DF] = acc * (1.0 / 3.0)


@functools.cache
def _make_mean():
    return pl.pallas_call(
        _mean_body,
        grid=(NP // _BN,),
        in_specs=[pl.BlockSpec((_BN, DF), lambda i: (i, 0))
                  for _ in range(3 * NF)],
        out_specs=pl.BlockSpec((_BN, EMB), lambda i: (i, 0)),
        out_shape=jax.ShapeDtypeStruct((NP, EMB), jnp.float32),
    )


def kernel(user_emb, item_emb, all_h_list, all_t_list):
    h = all_h_list
    t = all_t_list
    hpg = h + jnp.where(h >= NU, NPH - NU, 0).astype(jnp.int32)
    tpg = t + jnp.where(t >= NU, NPH - NU, 0).astype(jnp.int32)
    padn = EH - EH_RAW
    hpg2 = jnp.concatenate([
        hpg[:EH_RAW], jnp.full((padn,), NU, jnp.int32),
        hpg[EH_RAW:], jnp.full((padn,), NPH + NU, jnp.int32),
    ])
    tpg2 = jnp.concatenate([
        tpg[:EH_RAW], jnp.zeros((padn,), jnp.int32),
        tpg[EH_RAW:], jnp.zeros((padn,), jnp.int32),
    ])
    hl2 = jnp.concatenate([hpg2[:EH], hpg2[EH:] - NPH])

    zpad = jnp.zeros((NPH - NU, DF), jnp.float32)
    ego = [
        jnp.concatenate([user_emb[:, f * DF:(f + 1) * DF], zpad,
                         item_emb[:, f * DF:(f + 1) * DF], zpad], axis=0)
        for f in range(NF)
    ]

    dsum = _make_dsum()
    gather4 = _make_gather4()
    att = _make_att()
    msg = [_make_message(f) for f in range(NF)]

    a_val = jnp.ones((NF, E2), jnp.float32)
    all_layers = [ego]
    for layer in range(2):
        tails = gather4(tpg2, *ego)
        layer_f = None
        for it in range(2):
            d = dsum(a_val, hl2)
            layer_f = [msg[f](a_val, d[f], tails, hl2, tpg2)
                       for f in range(NF)]
            last = layer == 1 and it == 1
            if not last:
                heads = gather4(hpg2, *layer_f)
                a_val = att(a_val, heads, tails)
        ego = layer_f
        all_layers.append(ego)

    mean = _make_mean()
    emb = mean(*[tab for lay in all_layers for tab in lay])
    u_g = emb[:NU, :]
    i_g = emb[NPH:NPH + NI, :]
    return (u_g, i_g)


# speedup vs baseline: 11.6675x; 1.3405x over previous
import functools

import jax
import jax.numpy as jnp
from jax import lax
from jax.experimental import pallas as pl
from jax.experimental.pallas import tpu as pltpu
from jax.experimental.pallas import tpu_sc as plsc

NU = 25000
NI = 25000
NN = NU + NI
EMB = 128
NF = 4
DF = EMB // NF
E_RAW = 600000
EH_RAW = E_RAW // 2

NC = 2
NS = 16
CH = 128
EH = 307200
E2 = 2 * EH
EPW = EH // NS
NCHUNK = EPW // CH

NPH = 25088
NP = 2 * NPH
TPN = NPH // NS
ACC = NF * NPH
SLICE = ACC // NS


def _newton_rsqrt(x, iters=3):
    i = lax.bitcast_convert_type(x, jnp.int32)
    y = lax.bitcast_convert_type(jnp.int32(0x5F3759DF) - (i >> 1), jnp.float32)
    for _ in range(iters):
        y = y * (1.5 - 0.5 * x * y * y)
    return y


def _softmax4(abuf, j):
    a = [abuf[f, pl.ds(j * 16, 16)] for f in range(NF)]
    m = jnp.maximum(jnp.maximum(a[0], a[1]), jnp.maximum(a[2], a[3]))
    e = [jnp.exp(v - m) for v in a]
    r = 1.0 / (e[0] + e[1] + e[2] + e[3])
    return e, r


def _mesh():
    return plsc.VectorSubcoreMesh(core_axis_name="c", subcore_axis_name="s")


_SC_PARAMS = pltpu.CompilerParams(needs_layout_passes=False,
                                  use_tc_tiling_on_sc=False)


@functools.cache
def _make_dsum():
    @functools.partial(
        pl.kernel,
        mesh=_mesh(),
        compiler_params=_SC_PARAMS,
        out_type=tuple(jax.ShapeDtypeStruct((NP,), jnp.float32)
                       for _ in range(NF)),
        scratch_types=[
            pltpu.VMEM((NF, CH), jnp.float32),
            pltpu.VMEM((CH,), jnp.int32),
            pltpu.VMEM((CH,), jnp.float32),
            pltpu.VMEM((CH,), jnp.float32),
            pltpu.VMEM((CH,), jnp.float32),
            pltpu.VMEM((CH,), jnp.float32),
            pltpu.VMEM((TPN,), jnp.float32),
            pltpu.VMEM_SHARED((NPH,), jnp.float32),
            pltpu.VMEM_SHARED((NPH,), jnp.float32),
            pltpu.VMEM_SHARED((NPH,), jnp.float32),
            pltpu.VMEM_SHARED((NPH,), jnp.float32),
        ],
    )
    def dsum(a_hbm, hl_hbm, d0, d1, d2, d3, abuf, hbuf, tp0, tp1, tp2, tp3,
             lslice, acc0, acc1, acc2, acc3):
        d_out = [d0, d1, d2, d3]
        tpb = [tp0, tp1, tp2, tp3]
        accsh = [acc0, acc1, acc2, acc3]
        c = lax.axis_index("c")
        s = lax.axis_index("s")
        zero = jnp.zeros((16,), jnp.float32)

        def zbody(i, _):
            lslice[pl.ds(i * 16, 16)] = zero
            return 0

        lax.fori_loop(0, TPN // 16, zbody, 0)
        base_row = s * TPN
        for f in range(NF):
            pltpu.sync_copy(lslice, accsh[f].at[pl.ds(base_row, TPN)])
        plsc.subcore_barrier()

        def chunk(i, _):
            gbase = c * EH + s * EPW + i * CH
            pltpu.sync_copy(a_hbm.at[:, pl.ds(gbase, CH)], abuf)
            pltpu.sync_copy(hl_hbm.at[pl.ds(gbase, CH)], hbuf)
            for j in range(CH // 16):
                e, r = _softmax4(abuf, j)
                for f in range(NF):
                    tpb[f][pl.ds(j * 16, 16)] = e[f] * r
            for f in range(NF):
                pltpu.sync_copy(tpb[f], accsh[f].at[hbuf], add=True)
            return 0

        lax.fori_loop(0, NCHUNK, chunk, 0)
        plsc.subcore_barrier()

        for f in range(NF):
            pltpu.sync_copy(accsh[f].at[pl.ds(base_row, TPN)], lslice)

            def finb(i, _):
                x = jnp.maximum(lslice[pl.ds(i * 16, 16)], 1e-8)
                lslice[pl.ds(i * 16, 16)] = _newton_rsqrt(x)
                return 0

            lax.fori_loop(0, TPN // 16, finb, 0)
            pltpu.sync_copy(lslice,
                            d_out[f].at[pl.ds(c * NPH + base_row, TPN)])

    return dsum


BLK = 3
BLKC = BLK * CH
NBLK = EPW // BLKC
NROWS = E2 // CH


def _k2_scratch():
    per_buf = [
        pltpu.VMEM((NF, BLKC), jnp.float32),
        pltpu.VMEM((BLK, CH), jnp.int32),
        pltpu.VMEM((BLK, CH), jnp.int32),
        pltpu.VMEM((BLK, CH), jnp.int32),
        pltpu.VMEM((BLK, CH), jnp.int32),
        pltpu.VMEM((BLK, CH), jnp.float32),
        pltpu.VMEM((BLK, CH), jnp.float32),
        pltpu.VMEM((BLKC, DF), jnp.float32),
        pltpu.VMEM((BLKC, DF), jnp.float32),
        pltpu.SemaphoreType.DMA,
        pltpu.SemaphoreType.DMA,
        pltpu.SemaphoreType.DMA,
    ]
    return per_buf + per_buf + [pltpu.VMEM_SHARED((NPH, DF), jnp.float32)]


@functools.cache
def _make_message(f: int):
    @functools.partial(
        pl.kernel,
        mesh=_mesh(),
        compiler_params=_SC_PARAMS,
        out_type=jax.ShapeDtypeStruct((NP, DF), jnp.float32),
        scratch_types=_k2_scratch(),
    )
    def message(a_hbm, d_hbm, t_dense_hbm, hl2d_hbm, tpg2d_hbm, f_hbm,
                *scr):
        bufs = [dict(zip(
            ("ab", "hlb", "tpb", "hpg", "sid", "dh", "dt", "trow", "vrow",
             "semD", "semG", "semS"), scr[12 * b:12 * (b + 1)]))
            for b in range(2)]
        facc = scr[24]
        c = lax.axis_index("c")
        s = lax.axis_index("s")
        base_e = c * EH + s * EPW
        base_r = (c * (EH // CH) + s * (EPW // CH))

        def dense_pairs(blk, B):
            gbase = base_e + blk * BLKC
            rowb = base_r + blk * BLK
            return [
                (a_hbm.at[:, pl.ds(gbase, BLKC)], B["ab"]),
                (hl2d_hbm.at[pl.ds(rowb, BLK), :], B["hlb"]),
                (tpg2d_hbm.at[pl.ds(rowb, BLK), :], B["tpb"]),
                (t_dense_hbm.at[f, pl.ds(gbase, BLKC), :], B["trow"]),
            ]

        def fire_dense(blk, B):
            for src, dst in dense_pairs(blk, B):
                pltpu.async_copy(src, dst, B["semD"])

        def drain_dense(B):
            for src, dst in dense_pairs(0, B):
                pltpu.make_async_copy(src, dst, B["semD"]).wait()

        def hpgsid(B):
            for k in range(BLK):
                for j in range(CH // 16):
                    hv = B["hlb"][k, pl.ds(j * 16, 16)]
                    B["hpg"][k, pl.ds(j * 16, 16)] = hv + c * NPH
                    B["sid"][k, pl.ds(j * 16, 16)] = hv

        def fire_gathers(B):
            for k in range(BLK):
                pltpu.async_copy(d_hbm.at[B["hpg"].at[k]], B["dh"].at[k],
                                 B["semG"])
                pltpu.async_copy(d_hbm.at[B["tpb"].at[k]], B["dt"].at[k],
                                 B["semG"])

        def drain_gathers(B):
            for k in range(BLK):
                pltpu.make_async_copy(d_hbm.at[B["hpg"].at[k]],
                                      B["dh"].at[k], B["semG"]).wait()
                pltpu.make_async_copy(d_hbm.at[B["tpb"].at[k]],
                                      B["dt"].at[k], B["semG"]).wait()

        def fire_scatters(B):
            for k in range(BLK):
                pltpu.async_copy(B["vrow"].at[pl.ds(k * CH, CH)],
                                 facc.at[B["sid"].at[k]], B["semS"],
                                 add=True)

        def drain_scatters(B):
            for k in range(BLK):
                pltpu.make_async_copy(B["vrow"].at[pl.ds(k * CH, CH)],
                                      facc.at[B["sid"].at[k]],
                                      B["semS"]).wait()

        def compute(B):
            ab, trow, vrow = B["ab"], B["trow"], B["vrow"]
            for k in range(BLK):
                for j in range(CH // 16):
                    col = k * CH + j * 16
                    a = [ab[ff, pl.ds(col, 16)] for ff in range(NF)]
                    m = jnp.maximum(jnp.maximum(a[0], a[1]),
                                    jnp.maximum(a[2], a[3]))
                    e = [jnp.exp(v - m) for v in a]
                    tp = e[f] / (e[0] + e[1] + e[2] + e[3])
                    dh = B["dh"][k, pl.ds(j * 16, 16)]
                    dt = B["dt"][k, pl.ds(j * 16, 16)]
                    ew = tp * dh * dt
                    for j2 in range(16):
                        row = col + j2
                        w = ew[j2]
                        vrow[row, pl.ds(0, 16)] = trow[row, pl.ds(0, 16)] * w
                        vrow[row, pl.ds(16, 16)] = trow[row, pl.ds(16, 16)] * w

        zero = jnp.zeros((16,), jnp.float32)
        dummy = jnp.full((16,), NU, jnp.int32)
        for i in range(BLKC):
            bufs[0]["vrow"][i, pl.ds(0, 16)] = zero
            bufs[0]["vrow"][i, pl.ds(16, 16)] = zero
            bufs[1]["vrow"][i, pl.ds(0, 16)] = zero
            bufs[1]["vrow"][i, pl.ds(16, 16)] = zero
        for k in range(BLK):
            for j in range(CH // 16):
                bufs[1]["sid"][k, pl.ds(j * 16, 16)] = dummy
        base_row = s * TPN
        for b in range(TPN // BLKC):
            pltpu.sync_copy(bufs[0]["vrow"],
                            facc.at[pl.ds(base_row + b * BLKC, BLKC)])
        rem = TPN - (TPN // BLKC) * BLKC
        pltpu.sync_copy(bufs[0]["vrow"].at[pl.ds(0, rem)],
                        facc.at[pl.ds(base_row + (TPN // BLKC) * BLKC, rem)])
        plsc.subcore_barrier()

        fire_dense(0, bufs[0])
        fire_dense(1, bufs[1])
        fire_scatters(bufs[1])
        drain_dense(bufs[0])
        hpgsid(bufs[0])
        fire_gathers(bufs[0])

        def body(i2, _):
            for b in (0, 1):
                blk = 2 * i2 + b
                ob = 1 - b
                drain_gathers(bufs[b])
                compute(bufs[b])
                fire_scatters(bufs[b])
                fire_dense(jnp.minimum(blk + 2, NBLK - 1), bufs[b])
                drain_scatters(bufs[ob])
                drain_dense(bufs[ob])
                hpgsid(bufs[ob])
                fire_gathers(bufs[ob])
            return 0

        lax.fori_loop(0, NBLK // 2, body, 0)

        drain_gathers(bufs[0])
        drain_scatters(bufs[1])
        drain_dense(bufs[1])
        plsc.subcore_barrier()
        pltpu.sync_copy(facc.at[pl.ds(base_row, TPN)],
                        f_hbm.at[pl.ds(c * NPH + base_row, TPN)])

    return message


@functools.cache
def _make_gather4():
    @functools.partial(
        pl.kernel,
        mesh=_mesh(),
        compiler_params=_SC_PARAMS,
        out_type=jax.ShapeDtypeStruct((NF, E2, DF), jnp.float32),
        scratch_types=[
            pltpu.VMEM((CH,), jnp.int32),
            pltpu.VMEM((CH, DF), jnp.float32),
            pltpu.SemaphoreType.DMA,
        ],
    )
    def gather4(idx_hbm, tab0, tab1, tab2, tab3, out_hbm, ibuf, rbuf, sem):
        c = lax.axis_index("c")
        s = lax.axis_index("s")
        tabs = [tab0, tab1, tab2, tab3]
        for f in range(NF):
            def chunk(i, _, f=f):
                gbase = c * EH + s * EPW + i * CH
                pltpu.sync_copy(idx_hbm.at[pl.ds(gbase, CH)], ibuf)
                pltpu.async_copy(tabs[f].at[ibuf], rbuf, sem).wait()
                pltpu.sync_copy(rbuf, out_hbm.at[f, pl.ds(gbase, CH), :])
                return 0

            lax.fori_loop(0, NCHUNK, chunk, 0)

    return gather4


_BE = 1536


def _att_body(a_ref, g_ref, t_ref, o_ref):
    for f in range(NF):
        g = g_ref[f]
        t = t_ref[f]
        gn = g / jnp.maximum(jnp.sqrt(jnp.sum(g * g, axis=1, keepdims=True)),
                             1e-12)
        tn = t / jnp.maximum(jnp.sqrt(jnp.sum(t * t, axis=1, keepdims=True)),
                             1e-12)
        o_ref[f, :] = a_ref[f, :] + jnp.sum(gn * jnp.tanh(tn), axis=1)


@functools.cache
def _make_att():
    return pl.pallas_call(
        _att_body,
        grid=(E2 // _BE,),
        in_specs=[
            pl.BlockSpec((NF, _BE), lambda i: (0, i)),
            pl.BlockSpec((NF, _BE, DF), lambda i: (0, i, 0)),
            pl.BlockSpec((NF, _BE, DF), lambda i: (0, i, 0)),
        ],
        out_specs=pl.BlockSpec((NF, _BE), lambda i: (0, i)),
        out_shape=jax.ShapeDtypeStruct((NF, E2), jnp.float32),
    )


_BN = 1024


def _mean_body(*refs):
    ins, o_ref = refs[:-1], refs[-1]
    for f in range(NF):
        acc = ins[f][...] + ins[NF + f][...] + ins[2 * NF + f][...]
        o_ref[:, f * DF:(f + 1) * DF] = acc * (1.0 / 3.0)


@functools.cache
def _make_mean():
    return pl.pallas_call(
        _mean_body,
        grid=(NP // _BN,),
        in_specs=[pl.BlockSpec((_BN, DF), lambda i: (i, 0))
                  for _ in range(3 * NF)],
        out_specs=pl.BlockSpec((_BN, EMB), lambda i: (i, 0)),
        out_shape=jax.ShapeDtypeStruct((NP, EMB), jnp.float32),
    )


def kernel(user_emb, item_emb, all_h_list, all_t_list):
    h = all_h_list
    t = all_t_list
    hpg = h + jnp.where(h >= NU, NPH - NU, 0).astype(jnp.int32)
    tpg = t + jnp.where(t >= NU, NPH - NU, 0).astype(jnp.int32)
    padn = EH - EH_RAW
    hpg2 = jnp.concatenate([
        hpg[:EH_RAW], jnp.full((padn,), NU, jnp.int32),
        hpg[EH_RAW:], jnp.full((padn,), NPH + NU, jnp.int32),
    ])
    tpg2 = jnp.concatenate([
        tpg[:EH_RAW], jnp.zeros((padn,), jnp.int32),
        tpg[EH_RAW:], jnp.zeros((padn,), jnp.int32),
    ])
    hl2 = jnp.concatenate([hpg2[:EH], hpg2[EH:] - NPH])
    hl2d = hl2.reshape(E2 // CH, CH)
    tpg2d = tpg2.reshape(E2 // CH, CH)

    zpad = jnp.zeros((NPH - NU, DF), jnp.float32)
    ego = [
        jnp.concatenate([user_emb[:, f * DF:(f + 1) * DF], zpad,
                         item_emb[:, f * DF:(f + 1) * DF], zpad], axis=0)
        for f in range(NF)
    ]

    dsum = _make_dsum()
    gather4 = _make_gather4()
    att = _make_att()
    msg = [_make_message(f) for f in range(NF)]

    a_val = jnp.ones((NF, E2), jnp.float32)
    all_layers = [ego]
    for layer in range(2):
        tails = gather4(tpg2, *ego)
        layer_f = None
        for it in range(2):
            d = dsum(a_val, hl2)
            layer_f = [msg[f](a_val, d[f], tails, hl2d, tpg2d)
                       for f in range(NF)]
            last = layer == 1 and it == 1
            if not last:
                heads = gather4(hpg2, *layer_f)
                a_val = att(a_val, heads, tails)
        ego = layer_f
        all_layers.append(ego)

    mean = _make_mean()
    emb = mean(*[tab for lay in all_layers for tab in lay])
    u_g = emb[:NU, :]
    i_g = emb[NPH:NPH + NI, :]
    return (u_g, i_g)


# speedup vs baseline: 13.1029x; 1.1230x over previous
import functools

import jax
import jax.numpy as jnp
from jax import lax
from jax.experimental import pallas as pl
from jax.experimental.pallas import tpu as pltpu
from jax.experimental.pallas import tpu_sc as plsc

NU = 25000
NI = 25000
NN = NU + NI
EMB = 128
NF = 4
DF = EMB // NF
E_RAW = 600000
EH_RAW = E_RAW // 2

NC = 2
NS = 16
CH = 128
EH = 307200
E2 = 2 * EH
EPW = EH // NS
NCHUNK = EPW // CH

NPH = 25088
NP = 2 * NPH
TPN = NPH // NS
ACC = NF * NPH
SLICE = ACC // NS


def _newton_rsqrt(x, iters=3):
    i = lax.bitcast_convert_type(x, jnp.int32)
    y = lax.bitcast_convert_type(jnp.int32(0x5F3759DF) - (i >> 1), jnp.float32)
    for _ in range(iters):
        y = y * (1.5 - 0.5 * x * y * y)
    return y


def _softmax4(abuf, j):
    a = [abuf[f, pl.ds(j * 16, 16)] for f in range(NF)]
    m = jnp.maximum(jnp.maximum(a[0], a[1]), jnp.maximum(a[2], a[3]))
    e = [jnp.exp(v - m) for v in a]
    r = 1.0 / (e[0] + e[1] + e[2] + e[3])
    return e, r


def _mesh():
    return plsc.VectorSubcoreMesh(core_axis_name="c", subcore_axis_name="s")


_SC_PARAMS = pltpu.CompilerParams(needs_layout_passes=False,
                                  use_tc_tiling_on_sc=False)


@functools.cache
def _make_dsum():
    @functools.partial(
        pl.kernel,
        mesh=_mesh(),
        compiler_params=_SC_PARAMS,
        out_type=tuple(jax.ShapeDtypeStruct((NP,), jnp.float32)
                       for _ in range(NF)),
        scratch_types=[
            pltpu.VMEM((NF, 384), jnp.float32),
            pltpu.VMEM((3, CH), jnp.int32),
            pltpu.VMEM((3, CH), jnp.int32),
            pltpu.VMEM((NF * 3, CH), jnp.float32),
            pltpu.SemaphoreType.DMA,
            pltpu.SemaphoreType.DMA,
            pltpu.VMEM((NF, 384), jnp.float32),
            pltpu.VMEM((3, CH), jnp.int32),
            pltpu.VMEM((3, CH), jnp.int32),
            pltpu.VMEM((NF * 3, CH), jnp.float32),
            pltpu.SemaphoreType.DMA,
            pltpu.SemaphoreType.DMA,
            pltpu.VMEM((TPN,), jnp.float32),
            pltpu.VMEM_SHARED((NPH,), jnp.float32),
            pltpu.VMEM_SHARED((NPH,), jnp.float32),
            pltpu.VMEM_SHARED((NPH,), jnp.float32),
            pltpu.VMEM_SHARED((NPH,), jnp.float32),
        ],
    )
    def dsum(a_hbm, hl2d_hbm, d0, d1, d2, d3, *scr):
        d_out = [d0, d1, d2, d3]
        bufs = [dict(zip(("ab", "hlb", "sid", "tpb", "semD", "semS"),
                         scr[6 * b:6 * (b + 1)])) for b in range(2)]
        lslice = scr[12]
        accsh = list(scr[13:17])
        c = lax.axis_index("c")
        s = lax.axis_index("s")
        zero = jnp.zeros((16,), jnp.float32)
        base_e = c * EH + s * EPW
        base_r = c * (EH // CH) + s * (EPW // CH)

        def fire_dense(blk, B):
            pltpu.async_copy(a_hbm.at[:, pl.ds(base_e + blk * BLKC, BLKC)],
                             B["ab"], B["semD"])
            pltpu.async_copy(hl2d_hbm.at[pl.ds(base_r + blk * BLK, BLK), :],
                             B["hlb"], B["semD"])

        def drain_dense(B):
            pltpu.make_async_copy(a_hbm.at[:, pl.ds(base_e, BLKC)],
                                  B["ab"], B["semD"]).wait()
            pltpu.make_async_copy(hl2d_hbm.at[pl.ds(base_r, BLK), :],
                                  B["hlb"], B["semD"]).wait()

        def compute(B):
            for k in range(BLK):
                for j in range(CH // 16):
                    col = k * CH + j * 16
                    e, r = _softmax4(B["ab"], col // 16)
                    B["sid"][k, pl.ds(j * 16, 16)] = B["hlb"][k, pl.ds(j * 16, 16)]
                    for f in range(NF):
                        B["tpb"][f * BLK + k, pl.ds(j * 16, 16)] = e[f] * r

        def fire_scat(B):
            for f in range(NF):
                for k in range(BLK):
                    pltpu.async_copy(B["tpb"].at[f * BLK + k],
                                     accsh[f].at[B["sid"].at[k]], B["semS"],
                                     add=True)

        def drain_scat(B):
            for f in range(NF):
                for k in range(BLK):
                    pltpu.make_async_copy(B["tpb"].at[f * BLK + k],
                                          accsh[f].at[B["sid"].at[k]],
                                          B["semS"]).wait()

        def zbody(i, _):
            lslice[pl.ds(i * 16, 16)] = zero
            return 0

        lax.fori_loop(0, TPN // 16, zbody, 0)
        base_row = s * TPN
        for f in range(NF):
            pltpu.sync_copy(lslice, accsh[f].at[pl.ds(base_row, TPN)])
        dummy = jnp.full((16,), NU, jnp.int32)
        for k in range(BLK):
            for j in range(CH // 16):
                bufs[1]["sid"][k, pl.ds(j * 16, 16)] = dummy
                for f in range(NF):
                    bufs[1]["tpb"][f * BLK + k, pl.ds(j * 16, 16)] = zero
        plsc.subcore_barrier()

        fire_dense(0, bufs[0])
        fire_dense(1, bufs[1])
        fire_scat(bufs[1])

        def body(i2, _):
            for b in (0, 1):
                blk = 2 * i2 + b
                ob = 1 - b
                drain_dense(bufs[b])
                compute(bufs[b])
                fire_scat(bufs[b])
                fire_dense(jnp.minimum(blk + 2, NBLK - 1), bufs[b])
                drain_scat(bufs[ob])
            return 0

        lax.fori_loop(0, NBLK // 2, body, 0)
        drain_scat(bufs[1])
        drain_dense(bufs[0])
        drain_dense(bufs[1])
        plsc.subcore_barrier()

        for f in range(NF):
            pltpu.sync_copy(accsh[f].at[pl.ds(base_row, TPN)], lslice)

            def finb(i, _):
                x = jnp.maximum(lslice[pl.ds(i * 16, 16)], 1e-8)
                lslice[pl.ds(i * 16, 16)] = _newton_rsqrt(x)
                return 0

            lax.fori_loop(0, TPN // 16, finb, 0)
            pltpu.sync_copy(lslice,
                            d_out[f].at[pl.ds(c * NPH + base_row, TPN)])

    return dsum


BLK = 3
BLKC = BLK * CH
NBLK = EPW // BLKC
NROWS = E2 // CH


def _k2_scratch():
    per_buf = [
        pltpu.VMEM((NF, BLKC), jnp.float32),
        pltpu.VMEM((BLK, CH), jnp.int32),
        pltpu.VMEM((BLK, CH), jnp.int32),
        pltpu.VMEM((BLK, CH), jnp.int32),
        pltpu.VMEM((BLK, CH), jnp.int32),
        pltpu.VMEM((BLK, CH), jnp.float32),
        pltpu.VMEM((BLK, CH), jnp.float32),
        pltpu.VMEM((BLKC, DF), jnp.float32),
        pltpu.VMEM((BLKC, DF), jnp.float32),
        pltpu.SemaphoreType.DMA,
        pltpu.SemaphoreType.DMA,
        pltpu.SemaphoreType.DMA,
    ]
    return per_buf + per_buf + [pltpu.VMEM_SHARED((NPH, DF), jnp.float32)]


@functools.cache
def _make_message(f: int):
    @functools.partial(
        pl.kernel,
        mesh=_mesh(),
        compiler_params=_SC_PARAMS,
        out_type=jax.ShapeDtypeStruct((NP, DF), jnp.float32),
        scratch_types=_k2_scratch(),
    )
    def message(a_hbm, d_hbm, t_dense_hbm, hl2d_hbm, tpg2d_hbm, f_hbm,
                *scr):
        bufs = [dict(zip(
            ("ab", "hlb", "tpb", "hpg", "sid", "dh", "dt", "trow", "vrow",
             "semD", "semG", "semS"), scr[12 * b:12 * (b + 1)]))
            for b in range(2)]
        facc = scr[24]
        c = lax.axis_index("c")
        s = lax.axis_index("s")
        base_e = c * EH + s * EPW
        base_r = (c * (EH // CH) + s * (EPW // CH))

        def dense_pairs(blk, B):
            gbase = base_e + blk * BLKC
            rowb = base_r + blk * BLK
            return [
                (a_hbm.at[:, pl.ds(gbase, BLKC)], B["ab"]),
                (hl2d_hbm.at[pl.ds(rowb, BLK), :], B["hlb"]),
                (tpg2d_hbm.at[pl.ds(rowb, BLK), :], B["tpb"]),
                (t_dense_hbm.at[f, pl.ds(gbase, BLKC), :], B["trow"]),
            ]

        def fire_dense(blk, B):
            for src, dst in dense_pairs(blk, B):
                pltpu.async_copy(src, dst, B["semD"])

        def drain_dense(B):
            for src, dst in dense_pairs(0, B):
                pltpu.make_async_copy(src, dst, B["semD"]).wait()

        def hpgsid(B):
            for k in range(BLK):
                for j in range(CH // 16):
                    hv = B["hlb"][k, pl.ds(j * 16, 16)]
                    B["hpg"][k, pl.ds(j * 16, 16)] = hv + c * NPH
                    B["sid"][k, pl.ds(j * 16, 16)] = hv

        def fire_gathers(B):
            for k in range(BLK):
                pltpu.async_copy(d_hbm.at[B["hpg"].at[k]], B["dh"].at[k],
                                 B["semG"])
                pltpu.async_copy(d_hbm.at[B["tpb"].at[k]], B["dt"].at[k],
                                 B["semG"])

        def drain_gathers(B):
            for k in range(BLK):
                pltpu.make_async_copy(d_hbm.at[B["hpg"].at[k]],
                                      B["dh"].at[k], B["semG"]).wait()
                pltpu.make_async_copy(d_hbm.at[B["tpb"].at[k]],
                                      B["dt"].at[k], B["semG"]).wait()

        def fire_scatters(B):
            for k in range(BLK):
                pltpu.async_copy(B["vrow"].at[pl.ds(k * CH, CH)],
                                 facc.at[B["sid"].at[k]], B["semS"],
                                 add=True)

        def drain_scatters(B):
            for k in range(BLK):
                pltpu.make_async_copy(B["vrow"].at[pl.ds(k * CH, CH)],
                                      facc.at[B["sid"].at[k]],
                                      B["semS"]).wait()

        def compute(B):
            ab, trow, vrow = B["ab"], B["trow"], B["vrow"]
            for k in range(BLK):
                for j in range(CH // 16):
                    col = k * CH + j * 16
                    a = [ab[ff, pl.ds(col, 16)] for ff in range(NF)]
                    m = jnp.maximum(jnp.maximum(a[0], a[1]),
                                    jnp.maximum(a[2], a[3]))
                    e = [jnp.exp(v - m) for v in a]
                    tp = e[f] / (e[0] + e[1] + e[2] + e[3])
                    dh = B["dh"][k, pl.ds(j * 16, 16)]
                    dt = B["dt"][k, pl.ds(j * 16, 16)]
                    ew = tp * dh * dt
                    for j2 in range(16):
                        row = col + j2
                        w = ew[j2]
                        vrow[row, pl.ds(0, 16)] = trow[row, pl.ds(0, 16)] * w
                        vrow[row, pl.ds(16, 16)] = trow[row, pl.ds(16, 16)] * w

        zero = jnp.zeros((16,), jnp.float32)
        dummy = jnp.full((16,), NU, jnp.int32)
        for i in range(BLKC):
            bufs[0]["vrow"][i, pl.ds(0, 16)] = zero
            bufs[0]["vrow"][i, pl.ds(16, 16)] = zero
            bufs[1]["vrow"][i, pl.ds(0, 16)] = zero
            bufs[1]["vrow"][i, pl.ds(16, 16)] = zero
        for k in range(BLK):
            for j in range(CH // 16):
                bufs[1]["sid"][k, pl.ds(j * 16, 16)] = dummy
        base_row = s * TPN
        for b in range(TPN // BLKC):
            pltpu.sync_copy(bufs[0]["vrow"],
                            facc.at[pl.ds(base_row + b * BLKC, BLKC)])
        rem = TPN - (TPN // BLKC) * BLKC
        pltpu.sync_copy(bufs[0]["vrow"].at[pl.ds(0, rem)],
                        facc.at[pl.ds(base_row + (TPN // BLKC) * BLKC, rem)])
        plsc.subcore_barrier()

        fire_dense(0, bufs[0])
        fire_dense(1, bufs[1])
        fire_scatters(bufs[1])
        drain_dense(bufs[0])
        hpgsid(bufs[0])
        fire_gathers(bufs[0])

        def body(i2, _):
            for b in (0, 1):
                blk = 2 * i2 + b
                ob = 1 - b
                drain_gathers(bufs[b])
                compute(bufs[b])
                fire_scatters(bufs[b])
                fire_dense(jnp.minimum(blk + 2, NBLK - 1), bufs[b])
                drain_scatters(bufs[ob])
                drain_dense(bufs[ob])
                hpgsid(bufs[ob])
                fire_gathers(bufs[ob])
            return 0

        lax.fori_loop(0, NBLK // 2, body, 0)

        drain_gathers(bufs[0])
        drain_scatters(bufs[1])
        drain_dense(bufs[1])
        plsc.subcore_barrier()
        pltpu.sync_copy(facc.at[pl.ds(base_row, TPN)],
                        f_hbm.at[pl.ds(c * NPH + base_row, TPN)])

    return message


@functools.cache
def _make_gather4():
    per_buf = [
        pltpu.VMEM((BLK, CH), jnp.int32),
        pltpu.VMEM((BLKC, DF), jnp.float32),
        pltpu.SemaphoreType.DMA,
        pltpu.SemaphoreType.DMA,
        pltpu.SemaphoreType.DMA,
    ]

    @functools.partial(
        pl.kernel,
        mesh=_mesh(),
        compiler_params=_SC_PARAMS,
        out_type=jax.ShapeDtypeStruct((NF, E2, DF), jnp.float32),
        scratch_types=per_buf + per_buf,
    )
    def gather4(idx2d_hbm, tab0, tab1, tab2, tab3, out_hbm, *scr):
        bufs = [dict(zip(("ib", "rb", "semI", "semG", "semW"),
                         scr[5 * b:5 * (b + 1)])) for b in range(2)]
        tabs = [tab0, tab1, tab2, tab3]
        c = lax.axis_index("c")
        s = lax.axis_index("s")
        base_e = c * EH + s * EPW
        base_r = c * (EH // CH) + s * (EPW // CH)

        for f in range(NF):
            tab = tabs[f]

            def fire_idx(blk, B):
                pltpu.async_copy(idx2d_hbm.at[pl.ds(base_r + blk * BLK, BLK), :],
                                 B["ib"], B["semI"])

            def drain_idx(B):
                pltpu.make_async_copy(idx2d_hbm.at[pl.ds(base_r, BLK), :],
                                      B["ib"], B["semI"]).wait()

            def fire_g(B):
                for k in range(BLK):
                    pltpu.async_copy(tab.at[B["ib"].at[k]],
                                     B["rb"].at[pl.ds(k * CH, CH)], B["semG"])

            def drain_g(B):
                for k in range(BLK):
                    pltpu.make_async_copy(tab.at[B["ib"].at[k]],
                                          B["rb"].at[pl.ds(k * CH, CH)],
                                          B["semG"]).wait()

            def fire_w(blk, B):
                pltpu.async_copy(B["rb"],
                                 out_hbm.at[f, pl.ds(base_e + blk * BLKC, BLKC), :],
                                 B["semW"])

            def drain_w(B):
                pltpu.make_async_copy(B["rb"],
                                      out_hbm.at[f, pl.ds(base_e, BLKC), :],
                                      B["semW"]).wait()

            fire_idx(0, bufs[0])
            fire_idx(1, bufs[1])
            drain_idx(bufs[0])
            fire_g(bufs[0])
            fire_w(1, bufs[1])

            def body(i2, _):
                for b in (0, 1):
                    blk = 2 * i2 + b
                    ob = 1 - b
                    drain_g(bufs[b])
                    fire_w(blk, bufs[b])
                    fire_idx(jnp.minimum(blk + 2, NBLK - 1), bufs[b])
                    drain_w(bufs[ob])
                    drain_idx(bufs[ob])
                    fire_g(bufs[ob])
                return 0

            lax.fori_loop(0, NBLK // 2, body, 0)
            drain_g(bufs[0])
            drain_w(bufs[1])
            drain_idx(bufs[1])

    return gather4


_BE = 1536


def _att_body(a_ref, g_ref, t_ref, o_ref):
    for f in range(NF):
        g = g_ref[f]
        t = t_ref[f]
        gn = g / jnp.maximum(jnp.sqrt(jnp.sum(g * g, axis=1, keepdims=True)),
                             1e-12)
        tn = t / jnp.maximum(jnp.sqrt(jnp.sum(t * t, axis=1, keepdims=True)),
                             1e-12)
        o_ref[f, :] = a_ref[f, :] + jnp.sum(gn * jnp.tanh(tn), axis=1)


@functools.cache
def _make_att():
    return pl.pallas_call(
        _att_body,
        grid=(E2 // _BE,),
        in_specs=[
            pl.BlockSpec((NF, _BE), lambda i: (0, i)),
            pl.BlockSpec((NF, _BE, DF), lambda i: (0, i, 0)),
            pl.BlockSpec((NF, _BE, DF), lambda i: (0, i, 0)),
        ],
        out_specs=pl.BlockSpec((NF, _BE), lambda i: (0, i)),
        out_shape=jax.ShapeDtypeStruct((NF, E2), jnp.float32),
    )


_BN = 1024


def _mean_body(*refs):
    ins, o_ref = refs[:-1], refs[-1]
    for f in range(NF):
        acc = ins[f][...] + ins[NF + f][...] + ins[2 * NF + f][...]
        o_ref[:, f * DF:(f + 1) * DF] = acc * (1.0 / 3.0)


@functools.cache
def _make_mean():
    return pl.pallas_call(
        _mean_body,
        grid=(NP // _BN,),
        in_specs=[pl.BlockSpec((_BN, DF), lambda i: (i, 0))
                  for _ in range(3 * NF)],
        out_specs=pl.BlockSpec((_BN, EMB), lambda i: (i, 0)),
        out_shape=jax.ShapeDtypeStruct((NP, EMB), jnp.float32),
    )


def kernel(user_emb, item_emb, all_h_list, all_t_list):
    h = all_h_list
    t = all_t_list
    hpg = h + jnp.where(h >= NU, NPH - NU, 0).astype(jnp.int32)
    tpg = t + jnp.where(t >= NU, NPH - NU, 0).astype(jnp.int32)
    padn = EH - EH_RAW
    hpg2 = jnp.concatenate([
        hpg[:EH_RAW], jnp.full((padn,), NU, jnp.int32),
        hpg[EH_RAW:], jnp.full((padn,), NPH + NU, jnp.int32),
    ])
    tpg2 = jnp.concatenate([
        tpg[:EH_RAW], jnp.zeros((padn,), jnp.int32),
        tpg[EH_RAW:], jnp.zeros((padn,), jnp.int32),
    ])
    hl2 = jnp.concatenate([hpg2[:EH], hpg2[EH:] - NPH])
    hl2d = hl2.reshape(E2 // CH, CH)
    tpg2d = tpg2.reshape(E2 // CH, CH)
    hpg2d = hpg2.reshape(E2 // CH, CH)

    zpad = jnp.zeros((NPH - NU, DF), jnp.float32)
    ego = [
        jnp.concatenate([user_emb[:, f * DF:(f + 1) * DF], zpad,
                         item_emb[:, f * DF:(f + 1) * DF], zpad], axis=0)
        for f in range(NF)
    ]

    dsum = _make_dsum()
    gather4 = _make_gather4()
    att = _make_att()
    msg = [_make_message(f) for f in range(NF)]

    a_val = jnp.ones((NF, E2), jnp.float32)
    all_layers = [ego]
    for layer in range(2):
        tails = gather4(tpg2d, *ego)
        layer_f = None
        for it in range(2):
            d = dsum(a_val, hl2d)
            layer_f = [msg[f](a_val, d[f], tails, hl2d, tpg2d)
                       for f in range(NF)]
            last = layer == 1 and it == 1
            if not last:
                heads = gather4(hpg2d, *layer_f)
                a_val = att(a_val, heads, tails)
        ego = layer_f
        all_layers.append(ego)

    mean = _make_mean()
    emb = mean(*[tab for lay in all_layers for tab in lay])
    u_g = emb[:NU, :]
    i_g = emb[NPH:NPH + NI, :]
    return (u_g, i_g)


# speedup vs baseline: 14.6334x; 1.1168x over previous
import functools

import jax
import jax.numpy as jnp
from jax import lax
from jax.experimental import pallas as pl
from jax.experimental.pallas import tpu as pltpu
from jax.experimental.pallas import tpu_sc as plsc

NU = 25000
NI = 25000
NN = NU + NI
EMB = 128
NF = 4
DF = EMB // NF
E_RAW = 600000
EH_RAW = E_RAW // 2

NC = 2
NS = 16
CH = 128
EH = 307200
E2 = 2 * EH
EPW = EH // NS
NCHUNK = EPW // CH

NPH = 25088
NP = 2 * NPH
TPN = NPH // NS
ACC = NF * NPH
SLICE = ACC // NS


def _newton_rsqrt(x, iters=3):
    i = lax.bitcast_convert_type(x, jnp.int32)
    y = lax.bitcast_convert_type(jnp.int32(0x5F3759DF) - (i >> 1), jnp.float32)
    for _ in range(iters):
        y = y * (1.5 - 0.5 * x * y * y)
    return y


def _softmax4(abuf, j):
    a = [abuf[f, pl.ds(j * 16, 16)] for f in range(NF)]
    m = jnp.maximum(jnp.maximum(a[0], a[1]), jnp.maximum(a[2], a[3]))
    e = [jnp.exp(v - m) for v in a]
    r = 1.0 / (e[0] + e[1] + e[2] + e[3])
    return e, r


def _mesh():
    return plsc.VectorSubcoreMesh(core_axis_name="c", subcore_axis_name="s")


_SC_PARAMS = pltpu.CompilerParams(needs_layout_passes=False,
                                  use_tc_tiling_on_sc=False)


@functools.cache
def _make_dsum():
    @functools.partial(
        pl.kernel,
        mesh=_mesh(),
        compiler_params=_SC_PARAMS,
        out_type=tuple(jax.ShapeDtypeStruct((NP,), jnp.float32)
                       for _ in range(NF)),
        scratch_types=[
            pltpu.VMEM((NF, 384), jnp.float32),
            pltpu.VMEM((3, CH), jnp.int32),
            pltpu.VMEM((3, CH), jnp.int32),
            pltpu.VMEM((NF * 3, CH), jnp.float32),
            pltpu.SemaphoreType.DMA,
            pltpu.SemaphoreType.DMA,
            pltpu.VMEM((NF, 384), jnp.float32),
            pltpu.VMEM((3, CH), jnp.int32),
            pltpu.VMEM((3, CH), jnp.int32),
            pltpu.VMEM((NF * 3, CH), jnp.float32),
            pltpu.SemaphoreType.DMA,
            pltpu.SemaphoreType.DMA,
            pltpu.VMEM((TPN,), jnp.float32),
            pltpu.VMEM_SHARED((NPH,), jnp.float32),
            pltpu.VMEM_SHARED((NPH,), jnp.float32),
            pltpu.VMEM_SHARED((NPH,), jnp.float32),
            pltpu.VMEM_SHARED((NPH,), jnp.float32),
        ],
    )
    def dsum(a_hbm, hl2d_hbm, d0, d1, d2, d3, *scr):
        d_out = [d0, d1, d2, d3]
        bufs = [dict(zip(("ab", "hlb", "sid", "tpb", "semD", "semS"),
                         scr[6 * b:6 * (b + 1)])) for b in range(2)]
        lslice = scr[12]
        accsh = list(scr[13:17])
        c = lax.axis_index("c")
        s = lax.axis_index("s")
        zero = jnp.zeros((16,), jnp.float32)
        base_e = c * EH + s * EPW
        base_r = c * (EH // CH) + s * (EPW // CH)

        def fire_dense(blk, B):
            pltpu.async_copy(a_hbm.at[:, pl.ds(base_e + blk * BLKC, BLKC)],
                             B["ab"], B["semD"])
            pltpu.async_copy(hl2d_hbm.at[pl.ds(base_r + blk * BLK, BLK), :],
                             B["hlb"], B["semD"])

        def drain_dense(B):
            pltpu.make_async_copy(a_hbm.at[:, pl.ds(base_e, BLKC)],
                                  B["ab"], B["semD"]).wait()
            pltpu.make_async_copy(hl2d_hbm.at[pl.ds(base_r, BLK), :],
                                  B["hlb"], B["semD"]).wait()

        def compute(B):
            for k in range(BLK):
                for j in range(CH // 16):
                    col = k * CH + j * 16
                    e, r = _softmax4(B["ab"], col // 16)
                    B["sid"][k, pl.ds(j * 16, 16)] = B["hlb"][k, pl.ds(j * 16, 16)]
                    for f in range(NF):
                        B["tpb"][f * BLK + k, pl.ds(j * 16, 16)] = e[f] * r

        def fire_scat(B):
            for f in range(NF):
                for k in range(BLK):
                    pltpu.async_copy(B["tpb"].at[f * BLK + k],
                                     accsh[f].at[B["sid"].at[k]], B["semS"],
                                     add=True)

        def drain_scat(B):
            for f in range(NF):
                for k in range(BLK):
                    pltpu.make_async_copy(B["tpb"].at[f * BLK + k],
                                          accsh[f].at[B["sid"].at[k]],
                                          B["semS"]).wait()

        def zbody(i, _):
            lslice[pl.ds(i * 16, 16)] = zero
            return 0

        lax.fori_loop(0, TPN // 16, zbody, 0)
        base_row = s * TPN
        for f in range(NF):
            pltpu.sync_copy(lslice, accsh[f].at[pl.ds(base_row, TPN)])
        dummy = jnp.full((16,), NU, jnp.int32)
        for k in range(BLK):
            for j in range(CH // 16):
                bufs[1]["sid"][k, pl.ds(j * 16, 16)] = dummy
                for f in range(NF):
                    bufs[1]["tpb"][f * BLK + k, pl.ds(j * 16, 16)] = zero
        plsc.subcore_barrier()

        fire_dense(0, bufs[0])
        fire_dense(1, bufs[1])
        fire_scat(bufs[1])

        def body(i2, _):
            for b in (0, 1):
                blk = 2 * i2 + b
                ob = 1 - b
                drain_dense(bufs[b])
                compute(bufs[b])
                fire_scat(bufs[b])
                fire_dense(jnp.minimum(blk + 2, NBLK - 1), bufs[b])
                drain_scat(bufs[ob])
            return 0

        lax.fori_loop(0, NBLK // 2, body, 0)
        drain_scat(bufs[1])
        drain_dense(bufs[0])
        drain_dense(bufs[1])
        plsc.subcore_barrier()

        for f in range(NF):
            pltpu.sync_copy(accsh[f].at[pl.ds(base_row, TPN)], lslice)

            def finb(i, _):
                x = jnp.maximum(lslice[pl.ds(i * 16, 16)], 1e-8)
                lslice[pl.ds(i * 16, 16)] = _newton_rsqrt(x)
                return 0

            lax.fori_loop(0, TPN // 16, finb, 0)
            pltpu.sync_copy(lslice,
                            d_out[f].at[pl.ds(c * NPH + base_row, TPN)])

    return dsum


BLK = 3
BLKC = BLK * CH
NBLK = EPW // BLKC
NROWS = E2 // CH


def _k2_scratch():
    per_buf = [
        pltpu.VMEM((NF, BLKC), jnp.float32),
        pltpu.VMEM((BLK, CH), jnp.int32),
        pltpu.VMEM((BLK, CH), jnp.int32),
        pltpu.VMEM((BLK, CH), jnp.int32),
        pltpu.VMEM((BLK, CH), jnp.int32),
        pltpu.VMEM((BLK, CH), jnp.float32),
        pltpu.VMEM((BLK, CH), jnp.float32),
        pltpu.VMEM((BLKC, DF), jnp.float32),
        pltpu.VMEM((BLKC, DF), jnp.float32),
        pltpu.SemaphoreType.DMA,
        pltpu.SemaphoreType.DMA,
        pltpu.SemaphoreType.DMA,
    ]
    return per_buf + per_buf + [pltpu.VMEM_SHARED((NPH, DF), jnp.float32)]


@functools.cache
def _make_message(f: int):
    @functools.partial(
        pl.kernel,
        mesh=_mesh(),
        compiler_params=_SC_PARAMS,
        out_type=jax.ShapeDtypeStruct((NP, DF), jnp.float32),
        scratch_types=_k2_scratch(),
    )
    def message(a_hbm, d_hbm, t_dense_hbm, hl2d_hbm, tpg2d_hbm, f_hbm,
                *scr):
        bufs = [dict(zip(
            ("ab", "hlb", "tpb", "hpg", "sid", "dh", "dt", "trow", "vrow",
             "semD", "semG", "semS"), scr[12 * b:12 * (b + 1)]))
            for b in range(2)]
        facc = scr[24]
        c = lax.axis_index("c")
        s = lax.axis_index("s")
        base_e = c * EH + s * EPW
        base_r = (c * (EH // CH) + s * (EPW // CH))

        def dense_pairs(blk, B):
            gbase = base_e + blk * BLKC
            rowb = base_r + blk * BLK
            return [
                (a_hbm.at[:, pl.ds(gbase, BLKC)], B["ab"]),
                (hl2d_hbm.at[pl.ds(rowb, BLK), :], B["hlb"]),
                (tpg2d_hbm.at[pl.ds(rowb, BLK), :], B["tpb"]),
                (t_dense_hbm.at[pl.ds(gbase, BLKC),
                                pl.ds(f * DF, DF)], B["trow"]),
            ]

        def fire_dense(blk, B):
            for src, dst in dense_pairs(blk, B):
                pltpu.async_copy(src, dst, B["semD"])

        def drain_dense(B):
            for src, dst in dense_pairs(0, B):
                pltpu.make_async_copy(src, dst, B["semD"]).wait()

        def hpgsid(B):
            for k in range(BLK):
                for j in range(CH // 16):
                    hv = B["hlb"][k, pl.ds(j * 16, 16)]
                    B["hpg"][k, pl.ds(j * 16, 16)] = hv + c * NPH
                    B["sid"][k, pl.ds(j * 16, 16)] = hv

        def fire_gathers(B):
            for k in range(BLK):
                pltpu.async_copy(d_hbm.at[B["hpg"].at[k]], B["dh"].at[k],
                                 B["semG"])
                pltpu.async_copy(d_hbm.at[B["tpb"].at[k]], B["dt"].at[k],
                                 B["semG"])

        def drain_gathers(B):
            for k in range(BLK):
                pltpu.make_async_copy(d_hbm.at[B["hpg"].at[k]],
                                      B["dh"].at[k], B["semG"]).wait()
                pltpu.make_async_copy(d_hbm.at[B["tpb"].at[k]],
                                      B["dt"].at[k], B["semG"]).wait()

        def fire_scatters(B):
            for k in range(BLK):
                pltpu.async_copy(B["vrow"].at[pl.ds(k * CH, CH)],
                                 facc.at[B["sid"].at[k]], B["semS"],
                                 add=True)

        def drain_scatters(B):
            for k in range(BLK):
                pltpu.make_async_copy(B["vrow"].at[pl.ds(k * CH, CH)],
                                      facc.at[B["sid"].at[k]],
                                      B["semS"]).wait()

        def compute(B):
            ab, trow, vrow = B["ab"], B["trow"], B["vrow"]
            for k in range(BLK):
                for j in range(CH // 16):
                    col = k * CH + j * 16
                    a = [ab[ff, pl.ds(col, 16)] for ff in range(NF)]
                    m = jnp.maximum(jnp.maximum(a[0], a[1]),
                                    jnp.maximum(a[2], a[3]))
                    e = [jnp.exp(v - m) for v in a]
                    tp = e[f] / (e[0] + e[1] + e[2] + e[3])
                    dh = B["dh"][k, pl.ds(j * 16, 16)]
                    dt = B["dt"][k, pl.ds(j * 16, 16)]
                    ew = tp * dh * dt
                    for j2 in range(16):
                        row = col + j2
                        w = ew[j2]
                        vrow[row, pl.ds(0, 16)] = trow[row, pl.ds(0, 16)] * w
                        vrow[row, pl.ds(16, 16)] = trow[row, pl.ds(16, 16)] * w

        zero = jnp.zeros((16,), jnp.float32)
        dummy = jnp.full((16,), NU, jnp.int32)
        for i in range(BLKC):
            bufs[0]["vrow"][i, pl.ds(0, 16)] = zero
            bufs[0]["vrow"][i, pl.ds(16, 16)] = zero
            bufs[1]["vrow"][i, pl.ds(0, 16)] = zero
            bufs[1]["vrow"][i, pl.ds(16, 16)] = zero
        for k in range(BLK):
            for j in range(CH // 16):
                bufs[1]["sid"][k, pl.ds(j * 16, 16)] = dummy
        base_row = s * TPN
        for b in range(TPN // BLKC):
            pltpu.sync_copy(bufs[0]["vrow"],
                            facc.at[pl.ds(base_row + b * BLKC, BLKC)])
        rem = TPN - (TPN // BLKC) * BLKC
        pltpu.sync_copy(bufs[0]["vrow"].at[pl.ds(0, rem)],
                        facc.at[pl.ds(base_row + (TPN // BLKC) * BLKC, rem)])
        plsc.subcore_barrier()

        fire_dense(0, bufs[0])
        fire_dense(1, bufs[1])
        fire_scatters(bufs[1])
        drain_dense(bufs[0])
        hpgsid(bufs[0])
        fire_gathers(bufs[0])

        def body(i2, _):
            for b in (0, 1):
                blk = 2 * i2 + b
                ob = 1 - b
                drain_gathers(bufs[b])
                compute(bufs[b])
                fire_scatters(bufs[b])
                fire_dense(jnp.minimum(blk + 2, NBLK - 1), bufs[b])
                drain_scatters(bufs[ob])
                drain_dense(bufs[ob])
                hpgsid(bufs[ob])
                fire_gathers(bufs[ob])
            return 0

        lax.fori_loop(0, NBLK // 2, body, 0)

        drain_gathers(bufs[0])
        drain_scatters(bufs[1])
        drain_dense(bufs[1])
        plsc.subcore_barrier()
        pltpu.sync_copy(facc.at[pl.ds(base_row, TPN)],
                        f_hbm.at[pl.ds(c * NPH + base_row, TPN)])

    return message


@functools.cache
def _make_gather4():
    per_buf = [
        pltpu.VMEM((BLK, CH), jnp.int32),
        pltpu.VMEM((BLKC, DF), jnp.float32),
        pltpu.SemaphoreType.DMA,
        pltpu.SemaphoreType.DMA,
        pltpu.SemaphoreType.DMA,
    ]

    @functools.partial(
        pl.kernel,
        mesh=_mesh(),
        compiler_params=_SC_PARAMS,
        out_type=jax.ShapeDtypeStruct((E2, EMB), jnp.float32),
        scratch_types=per_buf + per_buf,
    )
    def gather4(idx2d_hbm, tab0, tab1, tab2, tab3, out_hbm, *scr):
        bufs = [dict(zip(("ib", "rb", "semI", "semG", "semW"),
                         scr[5 * b:5 * (b + 1)])) for b in range(2)]
        tabs = [tab0, tab1, tab2, tab3]
        c = lax.axis_index("c")
        s = lax.axis_index("s")
        base_e = c * EH + s * EPW
        base_r = c * (EH // CH) + s * (EPW // CH)

        for f in range(NF):
            tab = tabs[f]

            def fire_idx(blk, B):
                pltpu.async_copy(idx2d_hbm.at[pl.ds(base_r + blk * BLK, BLK), :],
                                 B["ib"], B["semI"])

            def drain_idx(B):
                pltpu.make_async_copy(idx2d_hbm.at[pl.ds(base_r, BLK), :],
                                      B["ib"], B["semI"]).wait()

            def fire_g(B):
                for k in range(BLK):
                    pltpu.async_copy(tab.at[B["ib"].at[k]],
                                     B["rb"].at[pl.ds(k * CH, CH)], B["semG"])

            def drain_g(B):
                for k in range(BLK):
                    pltpu.make_async_copy(tab.at[B["ib"].at[k]],
                                          B["rb"].at[pl.ds(k * CH, CH)],
                                          B["semG"]).wait()

            def fire_w(blk, B):
                pltpu.async_copy(B["rb"],
                                 out_hbm.at[pl.ds(base_e + blk * BLKC, BLKC),
                                            pl.ds(f * DF, DF)],
                                 B["semW"])

            def drain_w(B):
                pltpu.make_async_copy(B["rb"],
                                      out_hbm.at[pl.ds(base_e, BLKC),
                                                 pl.ds(f * DF, DF)],
                                      B["semW"]).wait()

            fire_idx(0, bufs[0])
            fire_idx(1, bufs[1])
            drain_idx(bufs[0])
            fire_g(bufs[0])
            fire_w(1, bufs[1])

            def body(i2, _):
                for b in (0, 1):
                    blk = 2 * i2 + b
                    ob = 1 - b
                    drain_g(bufs[b])
                    fire_w(blk, bufs[b])
                    fire_idx(jnp.minimum(blk + 2, NBLK - 1), bufs[b])
                    drain_w(bufs[ob])
                    drain_idx(bufs[ob])
                    fire_g(bufs[ob])
                return 0

            lax.fori_loop(0, NBLK // 2, body, 0)
            drain_g(bufs[0])
            drain_w(bufs[1])
            drain_idx(bufs[1])

    return gather4


_BE = 1536


def _att_body(a_ref, g_ref, t_ref, o_ref):
    for f in range(NF):
        g = g_ref[:, f * DF:(f + 1) * DF]
        t = t_ref[:, f * DF:(f + 1) * DF]
        gn = g / jnp.maximum(jnp.sqrt(jnp.sum(g * g, axis=1, keepdims=True)),
                             1e-12)
        tn = t / jnp.maximum(jnp.sqrt(jnp.sum(t * t, axis=1, keepdims=True)),
                             1e-12)
        o_ref[f, :] = a_ref[f, :] + jnp.sum(gn * jnp.tanh(tn), axis=1)


@functools.cache
def _make_att():
    return pl.pallas_call(
        _att_body,
        grid=(E2 // _BE,),
        in_specs=[
            pl.BlockSpec((NF, _BE), lambda i: (0, i)),
            pl.BlockSpec((_BE, EMB), lambda i: (i, 0)),
            pl.BlockSpec((_BE, EMB), lambda i: (i, 0)),
        ],
        out_specs=pl.BlockSpec((NF, _BE), lambda i: (0, i)),
        out_shape=jax.ShapeDtypeStruct((NF, E2), jnp.float32),
    )


_BN = 1024


def _mean_body(*refs):
    ins, o_ref = refs[:-1], refs[-1]
    for f in range(NF):
        acc = ins[f][...] + ins[NF + f][...] + ins[2 * NF + f][...]
        o_ref[:, f * DF:(f + 1) * DF] = acc * (1.0 / 3.0)


@functools.cache
def _make_mean():
    return pl.pallas_call(
        _mean_body,
        grid=(NP // _BN,),
        in_specs=[pl.BlockSpec((_BN, DF), lambda i: (i, 0))
                  for _ in range(3 * NF)],
        out_specs=pl.BlockSpec((_BN, EMB), lambda i: (i, 0)),
        out_shape=jax.ShapeDtypeStruct((NP, EMB), jnp.float32),
    )


def kernel(user_emb, item_emb, all_h_list, all_t_list):
    h = all_h_list
    t = all_t_list
    hpg = h + jnp.where(h >= NU, NPH - NU, 0).astype(jnp.int32)
    tpg = t + jnp.where(t >= NU, NPH - NU, 0).astype(jnp.int32)
    padn = EH - EH_RAW
    hpg2 = jnp.concatenate([
        hpg[:EH_RAW], jnp.full((padn,), NU, jnp.int32),
        hpg[EH_RAW:], jnp.full((padn,), NPH + NU, jnp.int32),
    ])
    tpg2 = jnp.concatenate([
        tpg[:EH_RAW], jnp.zeros((padn,), jnp.int32),
        tpg[EH_RAW:], jnp.zeros((padn,), jnp.int32),
    ])
    hl2 = jnp.concatenate([hpg2[:EH], hpg2[EH:] - NPH])
    hl2d = hl2.reshape(E2 // CH, CH)
    tpg2d = tpg2.reshape(E2 // CH, CH)
    hpg2d = hpg2.reshape(E2 // CH, CH)

    zpad = jnp.zeros((NPH - NU, DF), jnp.float32)
    ego = [
        jnp.concatenate([user_emb[:, f * DF:(f + 1) * DF], zpad,
                         item_emb[:, f * DF:(f + 1) * DF], zpad], axis=0)
        for f in range(NF)
    ]

    dsum = _make_dsum()
    gather4 = _make_gather4()
    att = _make_att()
    msg = [_make_message(f) for f in range(NF)]

    a_val = jnp.ones((NF, E2), jnp.float32)
    all_layers = [ego]
    for layer in range(2):
        tails = gather4(tpg2d, *ego)
        layer_f = None
        for it in range(2):
            d = dsum(a_val, hl2d)
            layer_f = [msg[f](a_val, d[f], tails, hl2d, tpg2d)
                       for f in range(NF)]
            last = layer == 1 and it == 1
            if not last:
                heads = gather4(hpg2d, *layer_f)
                a_val = att(a_val, heads, tails)
        ego = layer_f
        all_layers.append(ego)

    mean = _make_mean()
    emb = mean(*[tab for lay in all_layers for tab in lay])
    u_g = emb[:NU, :]
    i_g = emb[NPH:NPH + NI, :]
    return (u_g, i_g)


# speedup vs baseline: 24.7938x; 1.6943x over previous
import functools

import jax
import jax.numpy as jnp
from jax import lax
from jax.experimental import pallas as pl
from jax.experimental.pallas import tpu as pltpu
from jax.experimental.pallas import tpu_sc as plsc

NU = 25000
NI = 25000
NN = NU + NI
EMB = 128
NF = 4
DF = EMB // NF
E_RAW = 600000
EH_RAW = E_RAW // 2

NC = 2
NS = 16
CH = 128
EH = 307200
E2 = 2 * EH
EPW = EH // NS
NCHUNK = EPW // CH

NPH = 25088
NP = 2 * NPH
TPN = NPH // NS
ACC = NF * NPH
SLICE = ACC // NS


def _newton_rsqrt(x, iters=3):
    i = lax.bitcast_convert_type(x, jnp.int32)
    y = lax.bitcast_convert_type(jnp.int32(0x5F3759DF) - (i >> 1), jnp.float32)
    for _ in range(iters):
        y = y * (1.5 - 0.5 * x * y * y)
    return y


def _softmax4(abuf, j):
    a = [abuf[f, pl.ds(j * 16, 16)] for f in range(NF)]
    m = jnp.maximum(jnp.maximum(a[0], a[1]), jnp.maximum(a[2], a[3]))
    e = [jnp.exp(v - m) for v in a]
    r = 1.0 / (e[0] + e[1] + e[2] + e[3])
    return e, r


def _mesh():
    return plsc.VectorSubcoreMesh(core_axis_name="c", subcore_axis_name="s")


_SC_PARAMS = pltpu.CompilerParams(needs_layout_passes=False,
                                  use_tc_tiling_on_sc=False)


@functools.cache
def _make_dsum():
    @functools.partial(
        pl.kernel,
        mesh=_mesh(),
        compiler_params=_SC_PARAMS,
        out_type=tuple(jax.ShapeDtypeStruct((NP,), jnp.float32)
                       for _ in range(NF)),
        scratch_types=[
            pltpu.VMEM((NF, 384), jnp.float32),
            pltpu.VMEM((3, CH), jnp.int32),
            pltpu.VMEM((3, CH), jnp.int32),
            pltpu.VMEM((NF * 3, CH), jnp.float32),
            pltpu.SemaphoreType.DMA,
            pltpu.SemaphoreType.DMA,
            pltpu.VMEM((NF, 384), jnp.float32),
            pltpu.VMEM((3, CH), jnp.int32),
            pltpu.VMEM((3, CH), jnp.int32),
            pltpu.VMEM((NF * 3, CH), jnp.float32),
            pltpu.SemaphoreType.DMA,
            pltpu.SemaphoreType.DMA,
            pltpu.VMEM((TPN,), jnp.float32),
            pltpu.VMEM_SHARED((NPH,), jnp.float32),
            pltpu.VMEM_SHARED((NPH,), jnp.float32),
            pltpu.VMEM_SHARED((NPH,), jnp.float32),
            pltpu.VMEM_SHARED((NPH,), jnp.float32),
        ],
    )
    def dsum(a_hbm, hl2d_hbm, d0, d1, d2, d3, *scr):
        d_out = [d0, d1, d2, d3]
        bufs = [dict(zip(("ab", "hlb", "sid", "tpb", "semD", "semS"),
                         scr[6 * b:6 * (b + 1)])) for b in range(2)]
        lslice = scr[12]
        accsh = list(scr[13:17])
        c = lax.axis_index("c")
        s = lax.axis_index("s")
        zero = jnp.zeros((16,), jnp.float32)
        base_e = c * EH + s * EPW
        base_r = c * (EH // CH) + s * (EPW // CH)

        def fire_dense(blk, B):
            pltpu.async_copy(a_hbm.at[:, pl.ds(base_e + blk * BLKC, BLKC)],
                             B["ab"], B["semD"])
            pltpu.async_copy(hl2d_hbm.at[pl.ds(base_r + blk * BLK, BLK), :],
                             B["hlb"], B["semD"])

        def drain_dense(B):
            pltpu.make_async_copy(a_hbm.at[:, pl.ds(base_e, BLKC)],
                                  B["ab"], B["semD"]).wait()
            pltpu.make_async_copy(hl2d_hbm.at[pl.ds(base_r, BLK), :],
                                  B["hlb"], B["semD"]).wait()

        def compute(B):
            for k in range(BLK):
                for j in range(CH // 16):
                    col = k * CH + j * 16
                    e, r = _softmax4(B["ab"], col // 16)
                    B["sid"][k, pl.ds(j * 16, 16)] = B["hlb"][k, pl.ds(j * 16, 16)]
                    for f in range(NF):
                        B["tpb"][f * BLK + k, pl.ds(j * 16, 16)] = e[f] * r

        def fire_scat(B):
            for f in range(NF):
                for k in range(BLK):
                    pltpu.async_copy(B["tpb"].at[f * BLK + k],
                                     accsh[f].at[B["sid"].at[k]], B["semS"],
                                     add=True)

        def drain_scat(B):
            for f in range(NF):
                for k in range(BLK):
                    pltpu.make_async_copy(B["tpb"].at[f * BLK + k],
                                          accsh[f].at[B["sid"].at[k]],
                                          B["semS"]).wait()

        def zbody(i, _):
            lslice[pl.ds(i * 16, 16)] = zero
            return 0

        lax.fori_loop(0, TPN // 16, zbody, 0)
        base_row = s * TPN
        for f in range(NF):
            pltpu.sync_copy(lslice, accsh[f].at[pl.ds(base_row, TPN)])
        dummy = jnp.full((16,), NU, jnp.int32)
        for k in range(BLK):
            for j in range(CH // 16):
                bufs[1]["sid"][k, pl.ds(j * 16, 16)] = dummy
                for f in range(NF):
                    bufs[1]["tpb"][f * BLK + k, pl.ds(j * 16, 16)] = zero
        plsc.subcore_barrier()

        fire_dense(0, bufs[0])
        fire_dense(1, bufs[1])
        fire_scat(bufs[1])

        def body(i2, _):
            for b in (0, 1):
                blk = 2 * i2 + b
                ob = 1 - b
                drain_dense(bufs[b])
                compute(bufs[b])
                fire_scat(bufs[b])
                fire_dense(jnp.minimum(blk + 2, NBLK - 1), bufs[b])
                drain_scat(bufs[ob])
            return 0

        lax.fori_loop(0, NBLK // 2, body, 0)
        drain_scat(bufs[1])
        drain_dense(bufs[0])
        drain_dense(bufs[1])
        plsc.subcore_barrier()

        for f in range(NF):
            pltpu.sync_copy(accsh[f].at[pl.ds(base_row, TPN)], lslice)

            def finb(i, _):
                x = jnp.maximum(lslice[pl.ds(i * 16, 16)], 1e-8)
                lslice[pl.ds(i * 16, 16)] = _newton_rsqrt(x)
                return 0

            lax.fori_loop(0, TPN // 16, finb, 0)
            pltpu.sync_copy(lslice,
                            d_out[f].at[pl.ds(c * NPH + base_row, TPN)])

    return dsum


BLK = 3
BLKC = BLK * CH
NBLK = EPW // BLKC
NROWS = E2 // CH


def _k2_scratch():
    per_buf = [
        pltpu.VMEM((NF, BLKC), jnp.float32),
        pltpu.VMEM((BLK, CH), jnp.int32),
        pltpu.VMEM((BLK, CH), jnp.int32),
        pltpu.VMEM((BLK, CH), jnp.int32),
        pltpu.VMEM((BLK, CH), jnp.int32),
        pltpu.VMEM((BLK, CH), jnp.float32),
        pltpu.VMEM((BLK, CH), jnp.float32),
        pltpu.VMEM((BLKC, DF), jnp.float32),
        pltpu.VMEM((BLKC, DF), jnp.float32),
        pltpu.SemaphoreType.DMA,
        pltpu.SemaphoreType.DMA,
        pltpu.SemaphoreType.DMA,
    ]
    return per_buf + per_buf + [pltpu.VMEM_SHARED((NPH, DF), jnp.float32)]


@functools.cache
def _make_message(f: int):
    @functools.partial(
        pl.kernel,
        mesh=_mesh(),
        compiler_params=_SC_PARAMS,
        out_type=jax.ShapeDtypeStruct((NP, DF), jnp.float32),
        scratch_types=_k2_scratch(),
    )
    def message(a_hbm, d_hbm, t_dense_hbm, hl2d_hbm, tpg2d_hbm, f_hbm,
                *scr):
        bufs = [dict(zip(
            ("ab", "hlb", "tpb", "hpg", "sid", "dh", "dt", "trow", "vrow",
             "semD", "semG", "semS"), scr[12 * b:12 * (b + 1)]))
            for b in range(2)]
        facc = scr[24]
        c = lax.axis_index("c")
        s = lax.axis_index("s")
        base_e = c * EH + s * EPW
        base_r = (c * (EH // CH) + s * (EPW // CH))

        def dense_pairs(blk, B):
            gbase = base_e + blk * BLKC
            rowb = base_r + blk * BLK
            return [
                (a_hbm.at[:, pl.ds(gbase, BLKC)], B["ab"]),
                (hl2d_hbm.at[pl.ds(rowb, BLK), :], B["hlb"]),
                (tpg2d_hbm.at[pl.ds(rowb, BLK), :], B["tpb"]),
                (t_dense_hbm.at[pl.ds(gbase, BLKC),
                                pl.ds(f * DF, DF)], B["trow"]),
            ]

        def fire_dense(blk, B):
            for src, dst in dense_pairs(blk, B):
                pltpu.async_copy(src, dst, B["semD"])

        def drain_dense(B):
            for src, dst in dense_pairs(0, B):
                pltpu.make_async_copy(src, dst, B["semD"]).wait()

        def hpgsid(B):
            for k in range(BLK):
                for j in range(CH // 16):
                    hv = B["hlb"][k, pl.ds(j * 16, 16)]
                    B["hpg"][k, pl.ds(j * 16, 16)] = hv + c * NPH
                    B["sid"][k, pl.ds(j * 16, 16)] = hv

        def fire_gathers(B):
            for k in range(BLK):
                pltpu.async_copy(d_hbm.at[B["hpg"].at[k]], B["dh"].at[k],
                                 B["semG"])
                pltpu.async_copy(d_hbm.at[B["tpb"].at[k]], B["dt"].at[k],
                                 B["semG"])

        def drain_gathers(B):
            for k in range(BLK):
                pltpu.make_async_copy(d_hbm.at[B["hpg"].at[k]],
                                      B["dh"].at[k], B["semG"]).wait()
                pltpu.make_async_copy(d_hbm.at[B["tpb"].at[k]],
                                      B["dt"].at[k], B["semG"]).wait()

        def fire_scatters(B):
            for k in range(BLK):
                pltpu.async_copy(B["vrow"].at[pl.ds(k * CH, CH)],
                                 facc.at[B["sid"].at[k]], B["semS"],
                                 add=True)

        def drain_scatters(B):
            for k in range(BLK):
                pltpu.make_async_copy(B["vrow"].at[pl.ds(k * CH, CH)],
                                      facc.at[B["sid"].at[k]],
                                      B["semS"]).wait()

        def compute(B):
            ab, trow, vrow = B["ab"], B["trow"], B["vrow"]
            for k in range(BLK):
                for j in range(CH // 16):
                    col = k * CH + j * 16
                    a = [ab[ff, pl.ds(col, 16)] for ff in range(NF)]
                    m = jnp.maximum(jnp.maximum(a[0], a[1]),
                                    jnp.maximum(a[2], a[3]))
                    e = [jnp.exp(v - m) for v in a]
                    tp = e[f] / (e[0] + e[1] + e[2] + e[3])
                    dh = B["dh"][k, pl.ds(j * 16, 16)]
                    dt = B["dt"][k, pl.ds(j * 16, 16)]
                    ew = tp * dh * dt
                    for j2 in range(16):
                        row = col + j2
                        w = ew[j2]
                        vrow[row, pl.ds(0, 16)] = trow[row, pl.ds(0, 16)] * w
                        vrow[row, pl.ds(16, 16)] = trow[row, pl.ds(16, 16)] * w

        zero = jnp.zeros((16,), jnp.float32)
        dummy = jnp.full((16,), NU, jnp.int32)
        for i in range(BLKC):
            bufs[0]["vrow"][i, pl.ds(0, 16)] = zero
            bufs[0]["vrow"][i, pl.ds(16, 16)] = zero
            bufs[1]["vrow"][i, pl.ds(0, 16)] = zero
            bufs[1]["vrow"][i, pl.ds(16, 16)] = zero
        for k in range(BLK):
            for j in range(CH // 16):
                bufs[1]["sid"][k, pl.ds(j * 16, 16)] = dummy
        base_row = s * TPN
        for b in range(TPN // BLKC):
            pltpu.sync_copy(bufs[0]["vrow"],
                            facc.at[pl.ds(base_row + b * BLKC, BLKC)])
        rem = TPN - (TPN // BLKC) * BLKC
        pltpu.sync_copy(bufs[0]["vrow"].at[pl.ds(0, rem)],
                        facc.at[pl.ds(base_row + (TPN // BLKC) * BLKC, rem)])
        plsc.subcore_barrier()

        fire_dense(0, bufs[0])
        fire_dense(1, bufs[1])
        fire_scatters(bufs[1])
        drain_dense(bufs[0])
        hpgsid(bufs[0])
        fire_gathers(bufs[0])

        def body(i2, _):
            for b in (0, 1):
                blk = 2 * i2 + b
                ob = 1 - b
                drain_gathers(bufs[b])
                compute(bufs[b])
                fire_scatters(bufs[b])
                fire_dense(jnp.minimum(blk + 2, NBLK - 1), bufs[b])
                drain_scatters(bufs[ob])
                drain_dense(bufs[ob])
                hpgsid(bufs[ob])
                fire_gathers(bufs[ob])
            return 0

        lax.fori_loop(0, NBLK // 2, body, 0)

        drain_gathers(bufs[0])
        drain_scatters(bufs[1])
        drain_dense(bufs[1])
        plsc.subcore_barrier()
        pltpu.sync_copy(facc.at[pl.ds(base_row, TPN)],
                        f_hbm.at[pl.ds(c * NPH + base_row, TPN)])

    return message


@functools.cache
def _make_gather4():
    per_buf = [
        pltpu.VMEM((BLK, CH), jnp.int32),
        pltpu.VMEM((BLKC, DF), jnp.float32),
        pltpu.SemaphoreType.DMA,
        pltpu.SemaphoreType.DMA,
        pltpu.SemaphoreType.DMA,
    ]

    @functools.partial(
        pl.kernel,
        mesh=_mesh(),
        compiler_params=_SC_PARAMS,
        out_type=jax.ShapeDtypeStruct((E2, EMB), jnp.float32),
        scratch_types=per_buf + per_buf,
    )
    def gather4(idx2d_hbm, tab0, tab1, tab2, tab3, out_hbm, *scr):
        bufs = [dict(zip(("ib", "rb", "semI", "semG", "semW"),
                         scr[5 * b:5 * (b + 1)])) for b in range(2)]
        tabs = [tab0, tab1, tab2, tab3]
        c = lax.axis_index("c")
        s = lax.axis_index("s")
        base_e = c * EH + s * EPW
        base_r = c * (EH // CH) + s * (EPW // CH)

        for f in range(NF):
            tab = tabs[f]

            def fire_idx(blk, B):
                pltpu.async_copy(idx2d_hbm.at[pl.ds(base_r + blk * BLK, BLK), :],
                                 B["ib"], B["semI"])

            def drain_idx(B):
                pltpu.make_async_copy(idx2d_hbm.at[pl.ds(base_r, BLK), :],
                                      B["ib"], B["semI"]).wait()

            def fire_g(B):
                for k in range(BLK):
                    pltpu.async_copy(tab.at[B["ib"].at[k]],
                                     B["rb"].at[pl.ds(k * CH, CH)], B["semG"])

            def drain_g(B):
                for k in range(BLK):
                    pltpu.make_async_copy(tab.at[B["ib"].at[k]],
                                          B["rb"].at[pl.ds(k * CH, CH)],
                                          B["semG"]).wait()

            def fire_w(blk, B):
                pltpu.async_copy(B["rb"],
                                 out_hbm.at[pl.ds(base_e + blk * BLKC, BLKC),
                                            pl.ds(f * DF, DF)],
                                 B["semW"])

            def drain_w(B):
                pltpu.make_async_copy(B["rb"],
                                      out_hbm.at[pl.ds(base_e, BLKC),
                                                 pl.ds(f * DF, DF)],
                                      B["semW"]).wait()

            fire_idx(0, bufs[0])
            fire_idx(1, bufs[1])
            drain_idx(bufs[0])
            fire_g(bufs[0])
            fire_w(1, bufs[1])

            def body(i2, _):
                for b in (0, 1):
                    blk = 2 * i2 + b
                    ob = 1 - b
                    drain_g(bufs[b])
                    fire_w(blk, bufs[b])
                    fire_idx(jnp.minimum(blk + 2, NBLK - 1), bufs[b])
                    drain_w(bufs[ob])
                    drain_idx(bufs[ob])
                    fire_g(bufs[ob])
                return 0

            lax.fori_loop(0, NBLK // 2, body, 0)
            drain_g(bufs[0])
            drain_w(bufs[1])
            drain_idx(bufs[1])

    return gather4


_BE = 4096


def _att_body(a_ref, g_ref, t_ref, o_ref):
    G = g_ref[...]
    T = t_ref[...]
    M = (lax.broadcasted_iota(jnp.int32, (EMB, NF), 0) // DF
         == lax.broadcasted_iota(jnp.int32, (EMB, NF), 1)
         ).astype(jnp.float32)
    Mt = (lax.broadcasted_iota(jnp.int32, (NF, EMB), 1) // DF
          == lax.broadcasted_iota(jnp.int32, (NF, EMB), 0)
          ).astype(jnp.float32)

    def mm(x, y):
        return jax.lax.dot(x, y, preferred_element_type=jnp.float32)

    ginv = 1.0 / jnp.maximum(jnp.sqrt(mm(G * G, M)), 1e-12)
    tinv = 1.0 / jnp.maximum(jnp.sqrt(mm(T * T, M)), 1e-12)
    prod = (G * mm(ginv, Mt)) * jnp.tanh(T * mm(tinv, Mt))
    u4 = mm(prod, M)
    o_ref[...] = a_ref[...] + u4.T


@functools.cache
def _make_att():
    return pl.pallas_call(
        _att_body,
        grid=(E2 // _BE,),
        in_specs=[
            pl.BlockSpec((NF, _BE), lambda i: (0, i)),
            pl.BlockSpec((_BE, EMB), lambda i: (i, 0)),
            pl.BlockSpec((_BE, EMB), lambda i: (i, 0)),
        ],
        out_specs=pl.BlockSpec((NF, _BE), lambda i: (0, i)),
        out_shape=jax.ShapeDtypeStruct((NF, E2), jnp.float32),
    )


_BN = 1024


def _mean_body(*refs):
    ins, o_ref = refs[:-1], refs[-1]
    for f in range(NF):
        acc = ins[f][...] + ins[NF + f][...] + ins[2 * NF + f][...]
        o_ref[:, f * DF:(f + 1) * DF] = acc * (1.0 / 3.0)


@functools.cache
def _make_mean():
    return pl.pallas_call(
        _mean_body,
        grid=(NP // _BN,),
        in_specs=[pl.BlockSpec((_BN, DF), lambda i: (i, 0))
                  for _ in range(3 * NF)],
        out_specs=pl.BlockSpec((_BN, EMB), lambda i: (i, 0)),
        out_shape=jax.ShapeDtypeStruct((NP, EMB), jnp.float32),
    )


def kernel(user_emb, item_emb, all_h_list, all_t_list):
    h = all_h_list
    t = all_t_list
    hpg = h + jnp.where(h >= NU, NPH - NU, 0).astype(jnp.int32)
    tpg = t + jnp.where(t >= NU, NPH - NU, 0).astype(jnp.int32)
    padn = EH - EH_RAW
    hpg2 = jnp.concatenate([
        hpg[:EH_RAW], jnp.full((padn,), NU, jnp.int32),
        hpg[EH_RAW:], jnp.full((padn,), NPH + NU, jnp.int32),
    ])
    tpg2 = jnp.concatenate([
        tpg[:EH_RAW], jnp.zeros((padn,), jnp.int32),
        tpg[EH_RAW:], jnp.zeros((padn,), jnp.int32),
    ])
    hl2 = jnp.concatenate([hpg2[:EH], hpg2[EH:] - NPH])
    hl2d = hl2.reshape(E2 // CH, CH)
    tpg2d = tpg2.reshape(E2 // CH, CH)
    hpg2d = hpg2.reshape(E2 // CH, CH)

    zpad = jnp.zeros((NPH - NU, DF), jnp.float32)
    ego = [
        jnp.concatenate([user_emb[:, f * DF:(f + 1) * DF], zpad,
                         item_emb[:, f * DF:(f + 1) * DF], zpad], axis=0)
        for f in range(NF)
    ]

    dsum = _make_dsum()
    gather4 = _make_gather4()
    att = _make_att()
    msg = [_make_message(f) for f in range(NF)]

    a_val = jnp.ones((NF, E2), jnp.float32)
    all_layers = [ego]
    for layer in range(2):
        tails = gather4(tpg2d, *ego)
        layer_f = None
        for it in range(2):
            d = dsum(a_val, hl2d)
            layer_f = [msg[f](a_val, d[f], tails, hl2d, tpg2d)
                       for f in range(NF)]
            last = layer == 1 and it == 1
            if not last:
                heads = gather4(hpg2d, *layer_f)
                a_val = att(a_val, heads, tails)
        ego = layer_f
        all_layers.append(ego)

    mean = _make_mean()
    emb = mean(*[tab for lay in all_layers for tab in lay])
    u_g = emb[:NU, :]
    i_g = emb[NPH:NPH + NI, :]
    return (u_g, i_g)


# speedup vs baseline: 25.6350x; 1.0339x over previous
import functools

import jax
import jax.numpy as jnp
from jax import lax
from jax.experimental import pallas as pl
from jax.experimental.pallas import tpu as pltpu
from jax.experimental.pallas import tpu_sc as plsc

NU = 25000
NI = 25000
NN = NU + NI
EMB = 128
NF = 4
DF = EMB // NF
E_RAW = 600000
EH_RAW = E_RAW // 2

NC = 2
NS = 16
CH = 128
EH = 307200
E2 = 2 * EH
EPW = EH // NS
NCHUNK = EPW // CH

NPH = 25088
NP = 2 * NPH
TPN = NPH // NS
ACC = NF * NPH
SLICE = ACC // NS


def _newton_rsqrt(x, iters=3):
    i = lax.bitcast_convert_type(x, jnp.int32)
    y = lax.bitcast_convert_type(jnp.int32(0x5F3759DF) - (i >> 1), jnp.float32)
    for _ in range(iters):
        y = y * (1.5 - 0.5 * x * y * y)
    return y


def _softmax4(abuf, j):
    a = [abuf[f, pl.ds(j * 16, 16)] for f in range(NF)]
    m = jnp.maximum(jnp.maximum(a[0], a[1]), jnp.maximum(a[2], a[3]))
    e = [jnp.exp(v - m) for v in a]
    r = 1.0 / (e[0] + e[1] + e[2] + e[3])
    return e, r


def _mesh():
    return plsc.VectorSubcoreMesh(core_axis_name="c", subcore_axis_name="s")


_SC_PARAMS = pltpu.CompilerParams(needs_layout_passes=False,
                                  use_tc_tiling_on_sc=False)


@functools.cache
def _make_dsum():
    @functools.partial(
        pl.kernel,
        mesh=_mesh(),
        compiler_params=_SC_PARAMS,
        out_type=tuple(jax.ShapeDtypeStruct((NP,), jnp.float32)
                       for _ in range(NF)),
        scratch_types=[
            pltpu.VMEM((NF, 384), jnp.float32),
            pltpu.VMEM((3, CH), jnp.int32),
            pltpu.VMEM((3, CH), jnp.int32),
            pltpu.VMEM((NF * 3, CH), jnp.float32),
            pltpu.SemaphoreType.DMA,
            pltpu.SemaphoreType.DMA,
            pltpu.VMEM((NF, 384), jnp.float32),
            pltpu.VMEM((3, CH), jnp.int32),
            pltpu.VMEM((3, CH), jnp.int32),
            pltpu.VMEM((NF * 3, CH), jnp.float32),
            pltpu.SemaphoreType.DMA,
            pltpu.SemaphoreType.DMA,
            pltpu.VMEM((TPN,), jnp.float32),
            pltpu.VMEM_SHARED((NPH,), jnp.float32),
            pltpu.VMEM_SHARED((NPH,), jnp.float32),
            pltpu.VMEM_SHARED((NPH,), jnp.float32),
            pltpu.VMEM_SHARED((NPH,), jnp.float32),
        ],
    )
    def dsum(a_hbm, hl2d_hbm, d0, d1, d2, d3, *scr):
        d_out = [d0, d1, d2, d3]
        bufs = [dict(zip(("ab", "hlb", "sid", "tpb", "semD", "semS"),
                         scr[6 * b:6 * (b + 1)])) for b in range(2)]
        lslice = scr[12]
        accsh = list(scr[13:17])
        c = lax.axis_index("c")
        s = lax.axis_index("s")
        zero = jnp.zeros((16,), jnp.float32)
        base_e = c * EH + s * EPW
        base_r = c * (EH // CH) + s * (EPW // CH)

        def fire_dense(blk, B):
            pltpu.async_copy(a_hbm.at[:, pl.ds(base_e + blk * BLKC, BLKC)],
                             B["ab"], B["semD"])
            pltpu.async_copy(hl2d_hbm.at[pl.ds(base_r + blk * BLK, BLK), :],
                             B["hlb"], B["semD"])

        def drain_dense(B):
            pltpu.make_async_copy(a_hbm.at[:, pl.ds(base_e, BLKC)],
                                  B["ab"], B["semD"]).wait()
            pltpu.make_async_copy(hl2d_hbm.at[pl.ds(base_r, BLK), :],
                                  B["hlb"], B["semD"]).wait()

        def compute(B):
            for k in range(BLK):
                for j in range(CH // 16):
                    col = k * CH + j * 16
                    e, r = _softmax4(B["ab"], col // 16)
                    B["sid"][k, pl.ds(j * 16, 16)] = B["hlb"][k, pl.ds(j * 16, 16)]
                    for f in range(NF):
                        B["tpb"][f * BLK + k, pl.ds(j * 16, 16)] = e[f] * r

        def fire_scat(B):
            for f in range(NF):
                for k in range(BLK):
                    pltpu.async_copy(B["tpb"].at[f * BLK + k],
                                     accsh[f].at[B["sid"].at[k]], B["semS"],
                                     add=True)

        def drain_scat(B):
            for f in range(NF):
                for k in range(BLK):
                    pltpu.make_async_copy(B["tpb"].at[f * BLK + k],
                                          accsh[f].at[B["sid"].at[k]],
                                          B["semS"]).wait()

        def zbody(i, _):
            lslice[pl.ds(i * 16, 16)] = zero
            return 0

        lax.fori_loop(0, TPN // 16, zbody, 0)
        base_row = s * TPN
        for f in range(NF):
            pltpu.sync_copy(lslice, accsh[f].at[pl.ds(base_row, TPN)])
        dummy = jnp.full((16,), NU, jnp.int32)
        for k in range(BLK):
            for j in range(CH // 16):
                bufs[1]["sid"][k, pl.ds(j * 16, 16)] = dummy
                for f in range(NF):
                    bufs[1]["tpb"][f * BLK + k, pl.ds(j * 16, 16)] = zero
        plsc.subcore_barrier()

        fire_dense(0, bufs[0])
        fire_dense(1, bufs[1])
        fire_scat(bufs[1])

        def body(i2, _):
            for b in (0, 1):
                blk = 2 * i2 + b
                ob = 1 - b
                drain_dense(bufs[b])
                compute(bufs[b])
                fire_scat(bufs[b])
                fire_dense(jnp.minimum(blk + 2, NBLK - 1), bufs[b])
                drain_scat(bufs[ob])
            return 0

        lax.fori_loop(0, NBLK // 2, body, 0)
        drain_scat(bufs[1])
        drain_dense(bufs[0])
        drain_dense(bufs[1])
        plsc.subcore_barrier()

        for f in range(NF):
            pltpu.sync_copy(accsh[f].at[pl.ds(base_row, TPN)], lslice)

            def finb(i, _):
                x = jnp.maximum(lslice[pl.ds(i * 16, 16)], 1e-8)
                lslice[pl.ds(i * 16, 16)] = _newton_rsqrt(x)
                return 0

            lax.fori_loop(0, TPN // 16, finb, 0)
            pltpu.sync_copy(lslice,
                            d_out[f].at[pl.ds(c * NPH + base_row, TPN)])

    return dsum


BLK = 3
BLKC = BLK * CH
NBLK = EPW // BLKC
NROWS = E2 // CH


def _k2_scratch():
    per_buf = [
        pltpu.VMEM((NF, BLKC), jnp.float32),
        pltpu.VMEM((BLK, CH), jnp.int32),
        pltpu.VMEM((BLK, CH), jnp.int32),
        pltpu.VMEM((BLK, CH), jnp.int32),
        pltpu.VMEM((BLK, CH), jnp.int32),
        pltpu.VMEM((BLK, CH), jnp.float32),
        pltpu.VMEM((BLK, CH), jnp.float32),
        pltpu.VMEM((BLKC, DF), jnp.float32),
        pltpu.VMEM((BLKC, DF), jnp.float32),
        pltpu.SemaphoreType.DMA,
        pltpu.SemaphoreType.DMA,
        pltpu.SemaphoreType.DMA,
    ]
    return per_buf + per_buf + [pltpu.VMEM_SHARED((NPH, DF), jnp.float32)]


@functools.cache
def _make_message(f: int):
    @functools.partial(
        pl.kernel,
        mesh=_mesh(),
        compiler_params=_SC_PARAMS,
        out_type=jax.ShapeDtypeStruct((NP, DF), jnp.float32),
        scratch_types=_k2_scratch(),
    )
    def message(a_hbm, d_hbm, t_dense_hbm, hl2d_hbm, tpg2d_hbm, f_hbm,
                *scr):
        bufs = [dict(zip(
            ("ab", "hlb", "tpb", "hpg", "sid", "dh", "dt", "trow", "vrow",
             "semD", "semG", "semS"), scr[12 * b:12 * (b + 1)]))
            for b in range(2)]
        facc = scr[24]
        c = lax.axis_index("c")
        s = lax.axis_index("s")
        base_e = c * EH + s * EPW
        base_r = (c * (EH // CH) + s * (EPW // CH))

        def dense_pairs(blk, B):
            gbase = base_e + blk * BLKC
            rowb = base_r + blk * BLK
            return [
                (a_hbm.at[:, pl.ds(gbase, BLKC)], B["ab"]),
                (hl2d_hbm.at[pl.ds(rowb, BLK), :], B["hlb"]),
                (tpg2d_hbm.at[pl.ds(rowb, BLK), :], B["tpb"]),
                (t_dense_hbm.at[pl.ds(gbase, BLKC),
                                pl.ds(f * DF, DF)], B["trow"]),
            ]

        def fire_dense(blk, B):
            for src, dst in dense_pairs(blk, B):
                pltpu.async_copy(src, dst, B["semD"])

        def drain_dense(B):
            for src, dst in dense_pairs(0, B):
                pltpu.make_async_copy(src, dst, B["semD"]).wait()

        def hpgsid(B):
            for k in range(BLK):
                for j in range(CH // 16):
                    hv = B["hlb"][k, pl.ds(j * 16, 16)]
                    B["hpg"][k, pl.ds(j * 16, 16)] = hv + c * NPH
                    B["sid"][k, pl.ds(j * 16, 16)] = hv

        def fire_gathers(B):
            for k in range(BLK):
                pltpu.async_copy(d_hbm.at[B["hpg"].at[k]], B["dh"].at[k],
                                 B["semG"])
                pltpu.async_copy(d_hbm.at[B["tpb"].at[k]], B["dt"].at[k],
                                 B["semG"])

        def drain_gathers(B):
            for k in range(BLK):
                pltpu.make_async_copy(d_hbm.at[B["hpg"].at[k]],
                                      B["dh"].at[k], B["semG"]).wait()
                pltpu.make_async_copy(d_hbm.at[B["tpb"].at[k]],
                                      B["dt"].at[k], B["semG"]).wait()

        def fire_scatters(B):
            for k in range(BLK):
                pltpu.async_copy(B["vrow"].at[pl.ds(k * CH, CH)],
                                 facc.at[B["sid"].at[k]], B["semS"],
                                 add=True)

        def drain_scatters(B):
            for k in range(BLK):
                pltpu.make_async_copy(B["vrow"].at[pl.ds(k * CH, CH)],
                                      facc.at[B["sid"].at[k]],
                                      B["semS"]).wait()

        def compute(B):
            ab, trow, vrow = B["ab"], B["trow"], B["vrow"]
            for k in range(BLK):
                for j in range(CH // 16):
                    col = k * CH + j * 16
                    a = [ab[ff, pl.ds(col, 16)] for ff in range(NF)]
                    m = jnp.maximum(jnp.maximum(a[0], a[1]),
                                    jnp.maximum(a[2], a[3]))
                    e = [jnp.exp(v - m) for v in a]
                    tp = e[f] / (e[0] + e[1] + e[2] + e[3])
                    dh = B["dh"][k, pl.ds(j * 16, 16)]
                    dt = B["dt"][k, pl.ds(j * 16, 16)]
                    ew = tp * dh * dt
                    for j2 in range(16):
                        row = col + j2
                        w = ew[j2]
                        vrow[row, pl.ds(0, 16)] = trow[row, pl.ds(0, 16)] * w
                        vrow[row, pl.ds(16, 16)] = trow[row, pl.ds(16, 16)] * w

        zero = jnp.zeros((16,), jnp.float32)
        dummy = jnp.full((16,), NU, jnp.int32)
        for i in range(BLKC):
            bufs[0]["vrow"][i, pl.ds(0, 16)] = zero
            bufs[0]["vrow"][i, pl.ds(16, 16)] = zero
            bufs[1]["vrow"][i, pl.ds(0, 16)] = zero
            bufs[1]["vrow"][i, pl.ds(16, 16)] = zero
        for k in range(BLK):
            for j in range(CH // 16):
                bufs[1]["sid"][k, pl.ds(j * 16, 16)] = dummy
        base_row = s * TPN
        for b in range(TPN // BLKC):
            pltpu.sync_copy(bufs[0]["vrow"],
                            facc.at[pl.ds(base_row + b * BLKC, BLKC)])
        rem = TPN - (TPN // BLKC) * BLKC
        pltpu.sync_copy(bufs[0]["vrow"].at[pl.ds(0, rem)],
                        facc.at[pl.ds(base_row + (TPN // BLKC) * BLKC, rem)])
        plsc.subcore_barrier()

        fire_dense(0, bufs[0])
        fire_dense(1, bufs[1])
        fire_scatters(bufs[1])
        drain_dense(bufs[0])
        hpgsid(bufs[0])
        fire_gathers(bufs[0])

        def body(i2, _):
            for b in (0, 1):
                blk = 2 * i2 + b
                ob = 1 - b
                drain_gathers(bufs[b])
                compute(bufs[b])
                fire_scatters(bufs[b])
                fire_dense(jnp.minimum(blk + 2, NBLK - 1), bufs[b])
                drain_scatters(bufs[ob])
                drain_dense(bufs[ob])
                hpgsid(bufs[ob])
                fire_gathers(bufs[ob])
            return 0

        lax.fori_loop(0, NBLK // 2, body, 0)

        drain_gathers(bufs[0])
        drain_scatters(bufs[1])
        drain_dense(bufs[1])
        plsc.subcore_barrier()
        pltpu.sync_copy(facc.at[pl.ds(base_row, TPN)],
                        f_hbm.at[pl.ds(c * NPH + base_row, TPN)])

    return message


GBLK = 5
GBLKC = GBLK * CH
GNBLK = EPW // GBLKC


@functools.cache
def _make_gather4():
    scratch = [
        pltpu.VMEM((GBLK, CH), jnp.int32),
        pltpu.VMEM((GBLK, CH), jnp.int32),
        pltpu.VMEM((GBLKC, DF), jnp.float32),
        pltpu.VMEM((GBLKC, DF), jnp.float32),
        pltpu.VMEM((GBLKC, DF), jnp.float32),
        pltpu.SemaphoreType.DMA,
        pltpu.SemaphoreType.DMA,
        pltpu.SemaphoreType.DMA,
        pltpu.SemaphoreType.DMA,
        pltpu.SemaphoreType.DMA,
        pltpu.SemaphoreType.DMA,
        pltpu.SemaphoreType.DMA,
        pltpu.SemaphoreType.DMA,
    ]

    @functools.partial(
        pl.kernel,
        mesh=_mesh(),
        compiler_params=_SC_PARAMS,
        out_type=jax.ShapeDtypeStruct((E2, EMB), jnp.float32),
        scratch_types=scratch,
    )
    def gather4(idx2d_hbm, tab0, tab1, tab2, tab3, out_hbm, *scr):
        ib = list(scr[0:2])
        rb = list(scr[2:5])
        semI = list(scr[5:7])
        semG = list(scr[7:10])
        semW = list(scr[10:13])
        tabs = [tab0, tab1, tab2, tab3]
        c = lax.axis_index("c")
        s = lax.axis_index("s")
        base_e = c * EH + s * EPW
        base_r = c * (EH // CH) + s * (EPW // CH)

        for f in range(NF):
            tab = tabs[f]

            def fire_idx(blk, i):
                pltpu.async_copy(
                    idx2d_hbm.at[pl.ds(base_r + blk * GBLK, GBLK), :],
                    ib[i], semI[i])

            def drain_idx(i):
                pltpu.make_async_copy(idx2d_hbm.at[pl.ds(base_r, GBLK), :],
                                      ib[i], semI[i]).wait()

            def fire_g(i, r):
                for k in range(GBLK):
                    pltpu.async_copy(tab.at[ib[i].at[k]],
                                     rb[r].at[pl.ds(k * CH, CH)], semG[r])

            def drain_g(i, r):
                for k in range(GBLK):
                    pltpu.make_async_copy(tab.at[ib[i].at[k]],
                                          rb[r].at[pl.ds(k * CH, CH)],
                                          semG[r]).wait()

            def fire_w(blk, r):
                pltpu.async_copy(rb[r],
                                 out_hbm.at[pl.ds(base_e + blk * GBLKC, GBLKC),
                                            pl.ds(f * DF, DF)],
                                 semW[r])

            def drain_w(r):
                pltpu.make_async_copy(rb[r],
                                      out_hbm.at[pl.ds(base_e, GBLKC),
                                                 pl.ds(f * DF, DF)],
                                      semW[r]).wait()

            fire_idx(0, 0)
            fire_idx(1, 1)
            drain_idx(0)
            fire_g(0, 0)
            fire_w(1, 1)
            fire_w(2, 2)

            def body(i2, _):
                for k in range(6):
                    sec = 6 * i2 + k
                    ibi, ibn = k % 2, (k + 1) % 2
                    rbi, rbn = k % 3, (k + 1) % 3
                    drain_w(rbn)
                    drain_idx(ibn)
                    fire_g(ibn, rbn)
                    drain_g(ibi, rbi)
                    fire_w(sec, rbi)
                    fire_idx(jnp.minimum(sec + 2, GNBLK - 1), ibi)
                return 0

            lax.fori_loop(0, GNBLK // 6, body, 0)
            drain_g(0, 0)
            drain_w(1)
            drain_w(2)
            drain_idx(1)

    return gather4


_BE = 4096


def _att_body(a_ref, g_ref, t_ref, o_ref):
    G = g_ref[...]
    T = t_ref[...]
    M = (lax.broadcasted_iota(jnp.int32, (EMB, NF), 0) // DF
         == lax.broadcasted_iota(jnp.int32, (EMB, NF), 1)
         ).astype(jnp.float32)
    Mt = (lax.broadcasted_iota(jnp.int32, (NF, EMB), 1) // DF
          == lax.broadcasted_iota(jnp.int32, (NF, EMB), 0)
          ).astype(jnp.float32)

    def mm(x, y):
        return jax.lax.dot(x, y, preferred_element_type=jnp.float32)

    ginv = 1.0 / jnp.maximum(jnp.sqrt(mm(G * G, M)), 1e-12)
    tinv = 1.0 / jnp.maximum(jnp.sqrt(mm(T * T, M)), 1e-12)
    prod = (G * mm(ginv, Mt)) * jnp.tanh(T * mm(tinv, Mt))
    u4 = mm(prod, M)
    o_ref[...] = a_ref[...] + u4.T


@functools.cache
def _make_att():
    return pl.pallas_call(
        _att_body,
        grid=(E2 // _BE,),
        in_specs=[
            pl.BlockSpec((NF, _BE), lambda i: (0, i)),
            pl.BlockSpec((_BE, EMB), lambda i: (i, 0)),
            pl.BlockSpec((_BE, EMB), lambda i: (i, 0)),
        ],
        out_specs=pl.BlockSpec((NF, _BE), lambda i: (0, i)),
        out_shape=jax.ShapeDtypeStruct((NF, E2), jnp.float32),
    )


_BN = 1024


def _mean_body(*refs):
    ins, o_ref = refs[:-1], refs[-1]
    for f in range(NF):
        acc = ins[f][...] + ins[NF + f][...] + ins[2 * NF + f][...]
        o_ref[:, f * DF:(f + 1) * DF] = acc * (1.0 / 3.0)


@functools.cache
def _make_mean():
    return pl.pallas_call(
        _mean_body,
        grid=(NP // _BN,),
        in_specs=[pl.BlockSpec((_BN, DF), lambda i: (i, 0))
                  for _ in range(3 * NF)],
        out_specs=pl.BlockSpec((_BN, EMB), lambda i: (i, 0)),
        out_shape=jax.ShapeDtypeStruct((NP, EMB), jnp.float32),
    )


def kernel(user_emb, item_emb, all_h_list, all_t_list):
    h = all_h_list
    t = all_t_list
    hpg = h + jnp.where(h >= NU, NPH - NU, 0).astype(jnp.int32)
    tpg = t + jnp.where(t >= NU, NPH - NU, 0).astype(jnp.int32)
    padn = EH - EH_RAW
    hpg2 = jnp.concatenate([
        hpg[:EH_RAW], jnp.full((padn,), NU, jnp.int32),
        hpg[EH_RAW:], jnp.full((padn,), NPH + NU, jnp.int32),
    ])
    tpg2 = jnp.concatenate([
        tpg[:EH_RAW], jnp.zeros((padn,), jnp.int32),
        tpg[EH_RAW:], jnp.zeros((padn,), jnp.int32),
    ])
    hl2 = jnp.concatenate([hpg2[:EH], hpg2[EH:] - NPH])
    hl2d = hl2.reshape(E2 // CH, CH)
    tpg2d = tpg2.reshape(E2 // CH, CH)
    hpg2d = hpg2.reshape(E2 // CH, CH)

    zpad = jnp.zeros((NPH - NU, DF), jnp.float32)
    ego = [
        jnp.concatenate([user_emb[:, f * DF:(f + 1) * DF], zpad,
                         item_emb[:, f * DF:(f + 1) * DF], zpad], axis=0)
        for f in range(NF)
    ]

    dsum = _make_dsum()
    gather4 = _make_gather4()
    att = _make_att()
    msg = [_make_message(f) for f in range(NF)]

    a_val = jnp.ones((NF, E2), jnp.float32)
    all_layers = [ego]
    for layer in range(2):
        tails = gather4(tpg2d, *ego)
        layer_f = None
        for it in range(2):
            d = dsum(a_val, hl2d)
            layer_f = [msg[f](a_val, d[f], tails, hl2d, tpg2d)
                       for f in range(NF)]
            last = layer == 1 and it == 1
            if not last:
                heads = gather4(hpg2d, *layer_f)
                a_val = att(a_val, heads, tails)
        ego = layer_f
        all_layers.append(ego)

    mean = _make_mean()
    emb = mean(*[tab for lay in all_layers for tab in lay])
    u_g = emb[:NU, :]
    i_g = emb[NPH:NPH + NI, :]
    return (u_g, i_g)


# speedup vs baseline: 28.4174x; 1.1085x over previous
import functools

import jax
import jax.numpy as jnp
from jax import lax
from jax.experimental import pallas as pl
from jax.experimental.pallas import tpu as pltpu
from jax.experimental.pallas import tpu_sc as plsc

NU = 25000
NI = 25000
NN = NU + NI
EMB = 128
NF = 4
DF = EMB // NF
E_RAW = 600000
EH_RAW = E_RAW // 2

NC = 2
NS = 16
CH = 128
EH = 307200
E2 = 2 * EH
EPW = EH // NS
NCHUNK = EPW // CH

NPH = 25088
NP = 2 * NPH
TPN = NPH // NS
ACC = NF * NPH
SLICE = ACC // NS


def _newton_rsqrt(x, iters=3):
    i = lax.bitcast_convert_type(x, jnp.int32)
    y = lax.bitcast_convert_type(jnp.int32(0x5F3759DF) - (i >> 1), jnp.float32)
    for _ in range(iters):
        y = y * (1.5 - 0.5 * x * y * y)
    return y


def _softmax4(abuf, j):
    a = [abuf[f, pl.ds(j * 16, 16)] for f in range(NF)]
    m = jnp.maximum(jnp.maximum(a[0], a[1]), jnp.maximum(a[2], a[3]))
    e = [jnp.exp(v - m) for v in a]
    r = 1.0 / (e[0] + e[1] + e[2] + e[3])
    return e, r


def _mesh():
    return plsc.VectorSubcoreMesh(core_axis_name="c", subcore_axis_name="s")


_SC_PARAMS = pltpu.CompilerParams(needs_layout_passes=False,
                                  use_tc_tiling_on_sc=False)


@functools.cache
def _make_dsum():
    @functools.partial(
        pl.kernel,
        mesh=_mesh(),
        compiler_params=_SC_PARAMS,
        out_type=tuple(jax.ShapeDtypeStruct((NP,), jnp.float32)
                       for _ in range(NF)),
        scratch_types=[
            pltpu.VMEM((NF, 384), jnp.float32),
            pltpu.VMEM((3, CH), jnp.int32),
            pltpu.VMEM((3, CH), jnp.int32),
            pltpu.VMEM((NF * 3, CH), jnp.float32),
            pltpu.SemaphoreType.DMA,
            pltpu.SemaphoreType.DMA,
            pltpu.VMEM((NF, 384), jnp.float32),
            pltpu.VMEM((3, CH), jnp.int32),
            pltpu.VMEM((3, CH), jnp.int32),
            pltpu.VMEM((NF * 3, CH), jnp.float32),
            pltpu.SemaphoreType.DMA,
            pltpu.SemaphoreType.DMA,
            pltpu.VMEM((TPN,), jnp.float32),
            pltpu.VMEM_SHARED((NPH,), jnp.float32),
            pltpu.VMEM_SHARED((NPH,), jnp.float32),
            pltpu.VMEM_SHARED((NPH,), jnp.float32),
            pltpu.VMEM_SHARED((NPH,), jnp.float32),
        ],
    )
    def dsum(a_hbm, hl2d_hbm, d0, d1, d2, d3, *scr):
        d_out = [d0, d1, d2, d3]
        bufs = [dict(zip(("ab", "hlb", "sid", "tpb", "semD", "semS"),
                         scr[6 * b:6 * (b + 1)])) for b in range(2)]
        lslice = scr[12]
        accsh = list(scr[13:17])
        c = lax.axis_index("c")
        s = lax.axis_index("s")
        zero = jnp.zeros((16,), jnp.float32)
        base_e = c * EH + s * EPW
        base_r = c * (EH // CH) + s * (EPW // CH)

        def fire_dense(blk, B):
            pltpu.async_copy(a_hbm.at[:, pl.ds(base_e + blk * BLKC, BLKC)],
                             B["ab"], B["semD"])
            pltpu.async_copy(hl2d_hbm.at[pl.ds(base_r + blk * BLK, BLK), :],
                             B["hlb"], B["semD"])

        def drain_dense(B):
            pltpu.make_async_copy(a_hbm.at[:, pl.ds(base_e, BLKC)],
                                  B["ab"], B["semD"]).wait()
            pltpu.make_async_copy(hl2d_hbm.at[pl.ds(base_r, BLK), :],
                                  B["hlb"], B["semD"]).wait()

        def compute(B):
            for k in range(BLK):
                for j in range(CH // 16):
                    col = k * CH + j * 16
                    e, r = _softmax4(B["ab"], col // 16)
                    B["sid"][k, pl.ds(j * 16, 16)] = B["hlb"][k, pl.ds(j * 16, 16)]
                    for f in range(NF):
                        B["tpb"][f * BLK + k, pl.ds(j * 16, 16)] = e[f] * r

        def fire_scat(B):
            for f in range(NF):
                for k in range(BLK):
                    pltpu.async_copy(B["tpb"].at[f * BLK + k],
                                     accsh[f].at[B["sid"].at[k]], B["semS"],
                                     add=True)

        def drain_scat(B):
            for f in range(NF):
                for k in range(BLK):
                    pltpu.make_async_copy(B["tpb"].at[f * BLK + k],
                                          accsh[f].at[B["sid"].at[k]],
                                          B["semS"]).wait()

        def zbody(i, _):
            lslice[pl.ds(i * 16, 16)] = zero
            return 0

        lax.fori_loop(0, TPN // 16, zbody, 0)
        base_row = s * TPN
        for f in range(NF):
            pltpu.sync_copy(lslice, accsh[f].at[pl.ds(base_row, TPN)])
        dummy = jnp.full((16,), NU, jnp.int32)
        for k in range(BLK):
            for j in range(CH // 16):
                bufs[1]["sid"][k, pl.ds(j * 16, 16)] = dummy
                for f in range(NF):
                    bufs[1]["tpb"][f * BLK + k, pl.ds(j * 16, 16)] = zero
        plsc.subcore_barrier()

        fire_dense(0, bufs[0])
        fire_dense(1, bufs[1])
        fire_scat(bufs[1])

        def body(i2, _):
            for b in (0, 1):
                blk = 2 * i2 + b
                ob = 1 - b
                drain_dense(bufs[b])
                compute(bufs[b])
                fire_scat(bufs[b])
                fire_dense(jnp.minimum(blk + 2, NBLK - 1), bufs[b])
                drain_scat(bufs[ob])
            return 0

        lax.fori_loop(0, NBLK // 2, body, 0)
        drain_scat(bufs[1])
        drain_dense(bufs[0])
        drain_dense(bufs[1])
        plsc.subcore_barrier()

        for f in range(NF):
            pltpu.sync_copy(accsh[f].at[pl.ds(base_row, TPN)], lslice)

            def finb(i, _):
                x = jnp.maximum(lslice[pl.ds(i * 16, 16)], 1e-8)
                lslice[pl.ds(i * 16, 16)] = _newton_rsqrt(x)
                return 0

            lax.fori_loop(0, TPN // 16, finb, 0)
            pltpu.sync_copy(lslice,
                            d_out[f].at[pl.ds(c * NPH + base_row, TPN)])

    return dsum


BLK = 3
BLKC = BLK * CH
NBLK = EPW // BLKC
NROWS = E2 // CH


def _k2_scratch():
    per_buf = [
        pltpu.VMEM((NF, BLKC), jnp.float32),
        pltpu.VMEM((BLK, CH), jnp.int32),
        pltpu.VMEM((BLK, CH), jnp.int32),
        pltpu.VMEM((BLK, CH), jnp.int32),
        pltpu.VMEM((BLK, CH), jnp.int32),
        pltpu.VMEM((BLK, CH), jnp.float32),
        pltpu.VMEM((BLK, CH), jnp.float32),
        pltpu.VMEM((BLKC, DF), jnp.float32),
        pltpu.VMEM((BLKC, DF), jnp.float32),
        pltpu.SemaphoreType.DMA,
        pltpu.SemaphoreType.DMA,
        pltpu.SemaphoreType.DMA,
    ]
    return per_buf + per_buf + [pltpu.VMEM_SHARED((NPH, DF), jnp.float32)]


@functools.cache
def _make_message(f: int):
    @functools.partial(
        pl.kernel,
        mesh=_mesh(),
        compiler_params=_SC_PARAMS,
        out_type=jax.ShapeDtypeStruct((NP, DF), jnp.float32),
        scratch_types=_k2_scratch(),
    )
    def message(a_hbm, d_hbm, t_dense_hbm, hl2d_hbm, tpg2d_hbm, f_hbm,
                *scr):
        bufs = [dict(zip(
            ("ab", "hlb", "tpb", "hpg", "sid", "dh", "dt", "trow", "vrow",
             "semD", "semG", "semS"), scr[12 * b:12 * (b + 1)]))
            for b in range(2)]
        facc = scr[24]
        c = lax.axis_index("c")
        s = lax.axis_index("s")
        base_e = c * EH + s * EPW
        base_r = (c * (EH // CH) + s * (EPW // CH))

        def dense_pairs(blk, B):
            gbase = base_e + blk * BLKC
            rowb = base_r + blk * BLK
            return [
                (a_hbm.at[:, pl.ds(gbase, BLKC)], B["ab"]),
                (hl2d_hbm.at[pl.ds(rowb, BLK), :], B["hlb"]),
                (tpg2d_hbm.at[pl.ds(rowb, BLK), :], B["tpb"]),
                (t_dense_hbm.at[pl.ds(gbase, BLKC),
                                pl.ds(f * DF, DF)], B["trow"]),
            ]

        def fire_dense(blk, B):
            for src, dst in dense_pairs(blk, B):
                pltpu.async_copy(src, dst, B["semD"])

        def drain_dense(B):
            for src, dst in dense_pairs(0, B):
                pltpu.make_async_copy(src, dst, B["semD"]).wait()

        def hpgsid(B):
            for k in range(BLK):
                for j in range(CH // 16):
                    hv = B["hlb"][k, pl.ds(j * 16, 16)]
                    B["hpg"][k, pl.ds(j * 16, 16)] = hv + c * NPH
                    B["sid"][k, pl.ds(j * 16, 16)] = hv

        def fire_gathers(B):
            for k in range(BLK):
                pltpu.async_copy(d_hbm.at[B["hpg"].at[k]], B["dh"].at[k],
                                 B["semG"])
                pltpu.async_copy(d_hbm.at[B["tpb"].at[k]], B["dt"].at[k],
                                 B["semG"])

        def drain_gathers(B):
            for k in range(BLK):
                pltpu.make_async_copy(d_hbm.at[B["hpg"].at[k]],
                                      B["dh"].at[k], B["semG"]).wait()
                pltpu.make_async_copy(d_hbm.at[B["tpb"].at[k]],
                                      B["dt"].at[k], B["semG"]).wait()

        def fire_scatters(B):
            for k in range(BLK):
                pltpu.async_copy(B["vrow"].at[pl.ds(k * CH, CH)],
                                 facc.at[B["sid"].at[k]], B["semS"],
                                 add=True)

        def drain_scatters(B):
            for k in range(BLK):
                pltpu.make_async_copy(B["vrow"].at[pl.ds(k * CH, CH)],
                                      facc.at[B["sid"].at[k]],
                                      B["semS"]).wait()

        def compute(B):
            ab, trow, vrow = B["ab"], B["trow"], B["vrow"]
            for k in range(BLK):
                for j in range(CH // 16):
                    col = k * CH + j * 16
                    a = [ab[ff, pl.ds(col, 16)] for ff in range(NF)]
                    m = jnp.maximum(jnp.maximum(a[0], a[1]),
                                    jnp.maximum(a[2], a[3]))
                    e = [jnp.exp(v - m) for v in a]
                    tp = e[f] / (e[0] + e[1] + e[2] + e[3])
                    dh = B["dh"][k, pl.ds(j * 16, 16)]
                    dt = B["dt"][k, pl.ds(j * 16, 16)]
                    ew = tp * dh * dt
                    for j2 in range(16):
                        row = col + j2
                        w = ew[j2]
                        vrow[row, pl.ds(0, 16)] = trow[row, pl.ds(0, 16)] * w
                        vrow[row, pl.ds(16, 16)] = trow[row, pl.ds(16, 16)] * w

        zero = jnp.zeros((16,), jnp.float32)
        dummy = jnp.full((16,), NU, jnp.int32)
        for i in range(BLKC):
            bufs[0]["vrow"][i, pl.ds(0, 16)] = zero
            bufs[0]["vrow"][i, pl.ds(16, 16)] = zero
            bufs[1]["vrow"][i, pl.ds(0, 16)] = zero
            bufs[1]["vrow"][i, pl.ds(16, 16)] = zero
        for k in range(BLK):
            for j in range(CH // 16):
                bufs[1]["sid"][k, pl.ds(j * 16, 16)] = dummy
        base_row = s * TPN
        for b in range(TPN // BLKC):
            pltpu.sync_copy(bufs[0]["vrow"],
                            facc.at[pl.ds(base_row + b * BLKC, BLKC)])
        rem = TPN - (TPN // BLKC) * BLKC
        pltpu.sync_copy(bufs[0]["vrow"].at[pl.ds(0, rem)],
                        facc.at[pl.ds(base_row + (TPN // BLKC) * BLKC, rem)])
        plsc.subcore_barrier()

        fire_dense(0, bufs[0])
        fire_dense(1, bufs[1])
        fire_scatters(bufs[1])
        drain_dense(bufs[0])
        hpgsid(bufs[0])
        fire_gathers(bufs[0])

        def body(i2, _):
            for b in (0, 1):
                blk = 2 * i2 + b
                ob = 1 - b
                drain_gathers(bufs[b])
                compute(bufs[b])
                fire_scatters(bufs[b])
                fire_dense(jnp.minimum(blk + 2, NBLK - 1), bufs[b])
                drain_scatters(bufs[ob])
                drain_dense(bufs[ob])
                hpgsid(bufs[ob])
                fire_gathers(bufs[ob])
            return 0

        lax.fori_loop(0, NBLK // 2, body, 0)

        drain_gathers(bufs[0])
        drain_scatters(bufs[1])
        drain_dense(bufs[1])
        plsc.subcore_barrier()
        pltpu.sync_copy(facc.at[pl.ds(base_row, TPN)],
                        f_hbm.at[pl.ds(c * NPH + base_row, TPN)])

    return message


GBLK = 1
GBLKC = GBLK * CH
GNBLK = EPW // GBLKC


@functools.cache
def _make_gather4():
    scratch = [
        pltpu.VMEM((GBLK, CH), jnp.int32),
        pltpu.VMEM((GBLK, CH), jnp.int32),
        pltpu.VMEM((GBLKC, EMB), jnp.float32),
        pltpu.VMEM((GBLKC, EMB), jnp.float32),
        pltpu.VMEM((GBLKC, EMB), jnp.float32),
        pltpu.SemaphoreType.DMA,
        pltpu.SemaphoreType.DMA,
        pltpu.SemaphoreType.DMA,
        pltpu.SemaphoreType.DMA,
        pltpu.SemaphoreType.DMA,
        pltpu.SemaphoreType.DMA,
        pltpu.SemaphoreType.DMA,
        pltpu.SemaphoreType.DMA,
    ]

    @functools.partial(
        pl.kernel,
        mesh=_mesh(),
        compiler_params=_SC_PARAMS,
        out_type=jax.ShapeDtypeStruct((E2, EMB), jnp.float32),
        scratch_types=scratch,
    )
    def gather4(idx2d_hbm, tab_hbm, out_hbm, *scr):
        ib = list(scr[0:2])
        rb = list(scr[2:5])
        semI = list(scr[5:7])
        semG = list(scr[7:10])
        semW = list(scr[10:13])
        c = lax.axis_index("c")
        s = lax.axis_index("s")
        base_e = c * EH + s * EPW
        base_r = c * (EH // CH) + s * (EPW // CH)

        def fire_idx(blk, i):
            pltpu.async_copy(
                idx2d_hbm.at[pl.ds(base_r + blk * GBLK, GBLK), :],
                ib[i], semI[i])

        def drain_idx(i):
            pltpu.make_async_copy(idx2d_hbm.at[pl.ds(base_r, GBLK), :],
                                  ib[i], semI[i]).wait()

        def fire_g(i, r):
            for k in range(GBLK):
                pltpu.async_copy(tab_hbm.at[ib[i].at[k]],
                                 rb[r].at[pl.ds(k * CH, CH)], semG[r])

        def drain_g(i, r):
            for k in range(GBLK):
                pltpu.make_async_copy(tab_hbm.at[ib[i].at[k]],
                                      rb[r].at[pl.ds(k * CH, CH)],
                                      semG[r]).wait()

        def fire_w(blk, r):
            pltpu.async_copy(rb[r],
                             out_hbm.at[pl.ds(base_e + blk * GBLKC, GBLKC), :],
                             semW[r])

        def drain_w(r):
            pltpu.make_async_copy(rb[r],
                                  out_hbm.at[pl.ds(base_e, GBLKC), :],
                                  semW[r]).wait()

        fire_idx(0, 0)
        fire_idx(1, 1)
        drain_idx(0)
        fire_g(0, 0)
        fire_w(1, 1)
        fire_w(2, 2)

        def body(i2, _):
            for k in range(6):
                sec = 6 * i2 + k
                ibi, ibn = k % 2, (k + 1) % 2
                rbi, rbn = k % 3, (k + 1) % 3
                drain_w(rbn)
                drain_idx(ibn)
                fire_g(ibn, rbn)
                drain_g(ibi, rbi)
                fire_w(sec, rbi)
                fire_idx(jnp.minimum(sec + 2, GNBLK - 1), ibi)
            return 0

        lax.fori_loop(0, GNBLK // 6, body, 0)
        drain_g(0, 0)
        drain_w(1)
        drain_w(2)
        drain_idx(1)

    return gather4


_BN2 = 1024


def _pack_body(i0, i1, i2c, i3, o_ref):
    ins = (i0, i1, i2c, i3)
    for f in range(NF):
        o_ref[:, f * DF:(f + 1) * DF] = ins[f][...]


@functools.cache
def _make_pack4():
    return pl.pallas_call(
        _pack_body,
        grid=(NP // _BN2,),
        in_specs=[pl.BlockSpec((_BN2, DF), lambda i: (i, 0))
                  for _ in range(NF)],
        out_specs=pl.BlockSpec((_BN2, EMB), lambda i: (i, 0)),
        out_shape=jax.ShapeDtypeStruct((NP, EMB), jnp.float32),
    )


_BE = 4096


def _att_body(a_ref, g_ref, t_ref, o_ref):
    G = g_ref[...]
    T = t_ref[...]
    M = (lax.broadcasted_iota(jnp.int32, (EMB, NF), 0) // DF
         == lax.broadcasted_iota(jnp.int32, (EMB, NF), 1)
         ).astype(jnp.float32)
    Mt = (lax.broadcasted_iota(jnp.int32, (NF, EMB), 1) // DF
          == lax.broadcasted_iota(jnp.int32, (NF, EMB), 0)
          ).astype(jnp.float32)

    def mm(x, y):
        return jax.lax.dot(x, y, preferred_element_type=jnp.float32)

    ginv = 1.0 / jnp.maximum(jnp.sqrt(mm(G * G, M)), 1e-12)
    tinv = 1.0 / jnp.maximum(jnp.sqrt(mm(T * T, M)), 1e-12)
    prod = (G * mm(ginv, Mt)) * jnp.tanh(T * mm(tinv, Mt))
    u4 = mm(prod, M)
    o_ref[...] = a_ref[...] + u4.T


@functools.cache
def _make_att():
    return pl.pallas_call(
        _att_body,
        grid=(E2 // _BE,),
        in_specs=[
            pl.BlockSpec((NF, _BE), lambda i: (0, i)),
            pl.BlockSpec((_BE, EMB), lambda i: (i, 0)),
            pl.BlockSpec((_BE, EMB), lambda i: (i, 0)),
        ],
        out_specs=pl.BlockSpec((NF, _BE), lambda i: (0, i)),
        out_shape=jax.ShapeDtypeStruct((NF, E2), jnp.float32),
    )


_BN = 1024


def _mean_body(a_ref, b_ref, c_ref, o_ref):
    o_ref[...] = (a_ref[...] + b_ref[...] + c_ref[...]) * (1.0 / 3.0)


@functools.cache
def _make_mean():
    return pl.pallas_call(
        _mean_body,
        grid=(NP // _BN,),
        in_specs=[pl.BlockSpec((_BN, EMB), lambda i: (i, 0))
                  for _ in range(3)],
        out_specs=pl.BlockSpec((_BN, EMB), lambda i: (i, 0)),
        out_shape=jax.ShapeDtypeStruct((NP, EMB), jnp.float32),
    )


def kernel(user_emb, item_emb, all_h_list, all_t_list):
    h = all_h_list
    t = all_t_list
    hpg = h + jnp.where(h >= NU, NPH - NU, 0).astype(jnp.int32)
    tpg = t + jnp.where(t >= NU, NPH - NU, 0).astype(jnp.int32)
    padn = EH - EH_RAW
    hpg2 = jnp.concatenate([
        hpg[:EH_RAW], jnp.full((padn,), NU, jnp.int32),
        hpg[EH_RAW:], jnp.full((padn,), NPH + NU, jnp.int32),
    ])
    tpg2 = jnp.concatenate([
        tpg[:EH_RAW], jnp.zeros((padn,), jnp.int32),
        tpg[EH_RAW:], jnp.zeros((padn,), jnp.int32),
    ])
    hl2 = jnp.concatenate([hpg2[:EH], hpg2[EH:] - NPH])
    hl2d = hl2.reshape(E2 // CH, CH)
    tpg2d = tpg2.reshape(E2 // CH, CH)
    hpg2d = hpg2.reshape(E2 // CH, CH)

    zpad = jnp.zeros((NPH - NU, EMB), jnp.float32)
    ego = jnp.concatenate([user_emb, zpad, item_emb, zpad], axis=0)

    dsum = _make_dsum()
    gather4 = _make_gather4()
    att = _make_att()
    pack4 = _make_pack4()
    msg = [_make_message(f) for f in range(NF)]

    a_val = jnp.ones((NF, E2), jnp.float32)
    all_layers = [ego]
    for layer in range(2):
        tails = gather4(tpg2d, ego)
        fpack = None
        for it in range(2):
            d = dsum(a_val, hl2d)
            layer_f = [msg[f](a_val, d[f], tails, hl2d, tpg2d)
                       for f in range(NF)]
            fpack = pack4(*layer_f)
            last = layer == 1 and it == 1
            if not last:
                heads = gather4(hpg2d, fpack)
                a_val = att(a_val, heads, tails)
        ego = fpack
        all_layers.append(ego)

    mean = _make_mean()
    emb = mean(*all_layers)
    u_g = emb[:NU, :]
    i_g = emb[NPH:NPH + NI, :]
    return (u_g, i_g)


# speedup vs baseline: 28.4993x; 1.0029x over previous
import functools

import jax
import jax.numpy as jnp
from jax import lax
from jax.experimental import pallas as pl
from jax.experimental.pallas import tpu as pltpu
from jax.experimental.pallas import tpu_sc as plsc

NU = 25000
NI = 25000
NN = NU + NI
EMB = 128
NF = 4
DF = EMB // NF
E_RAW = 600000
EH_RAW = E_RAW // 2

NC = 2
NS = 16
CH = 128
EH = 307200
E2 = 2 * EH
EPW = EH // NS
NCHUNK = EPW // CH

NPH = 25088
NP = 2 * NPH
TPN = NPH // NS
ACC = NF * NPH
SLICE = ACC // NS


def _newton_rsqrt(x, iters=3):
    i = lax.bitcast_convert_type(x, jnp.int32)
    y = lax.bitcast_convert_type(jnp.int32(0x5F3759DF) - (i >> 1), jnp.float32)
    for _ in range(iters):
        y = y * (1.5 - 0.5 * x * y * y)
    return y


def _softmax4(abuf, j):
    a = [abuf[f, pl.ds(j * 16, 16)] for f in range(NF)]
    m = jnp.maximum(jnp.maximum(a[0], a[1]), jnp.maximum(a[2], a[3]))
    e = [jnp.exp(v - m) for v in a]
    r = 1.0 / (e[0] + e[1] + e[2] + e[3])
    return e, r


def _mesh():
    return plsc.VectorSubcoreMesh(core_axis_name="c", subcore_axis_name="s")


_SC_PARAMS = pltpu.CompilerParams(needs_layout_passes=False,
                                  use_tc_tiling_on_sc=False)


@functools.cache
def _make_dsum():
    @functools.partial(
        pl.kernel,
        mesh=_mesh(),
        compiler_params=_SC_PARAMS,
        out_type=tuple(jax.ShapeDtypeStruct((NP,), jnp.float32)
                       for _ in range(NF)),
        scratch_types=[
            pltpu.VMEM((NF, 384), jnp.float32),
            pltpu.VMEM((3, CH), jnp.int32),
            pltpu.VMEM((3, CH), jnp.int32),
            pltpu.VMEM((NF * 3, CH), jnp.float32),
            pltpu.SemaphoreType.DMA,
            pltpu.SemaphoreType.DMA,
            pltpu.VMEM((NF, 384), jnp.float32),
            pltpu.VMEM((3, CH), jnp.int32),
            pltpu.VMEM((3, CH), jnp.int32),
            pltpu.VMEM((NF * 3, CH), jnp.float32),
            pltpu.SemaphoreType.DMA,
            pltpu.SemaphoreType.DMA,
            pltpu.VMEM((TPN,), jnp.float32),
            pltpu.VMEM_SHARED((NPH,), jnp.float32),
            pltpu.VMEM_SHARED((NPH,), jnp.float32),
            pltpu.VMEM_SHARED((NPH,), jnp.float32),
            pltpu.VMEM_SHARED((NPH,), jnp.float32),
        ],
    )
    def dsum(a_hbm, hl2d_hbm, d0, d1, d2, d3, *scr):
        d_out = [d0, d1, d2, d3]
        bufs = [dict(zip(("ab", "hlb", "sid", "tpb", "semD", "semS"),
                         scr[6 * b:6 * (b + 1)])) for b in range(2)]
        lslice = scr[12]
        accsh = list(scr[13:17])
        c = lax.axis_index("c")
        s = lax.axis_index("s")
        zero = jnp.zeros((16,), jnp.float32)
        base_e = c * EH + s * EPW
        base_r = c * (EH // CH) + s * (EPW // CH)

        def fire_dense(blk, B):
            pltpu.async_copy(a_hbm.at[:, pl.ds(base_e + blk * BLKC, BLKC)],
                             B["ab"], B["semD"])
            pltpu.async_copy(hl2d_hbm.at[pl.ds(base_r + blk * BLK, BLK), :],
                             B["hlb"], B["semD"])

        def drain_dense(B):
            pltpu.make_async_copy(a_hbm.at[:, pl.ds(base_e, BLKC)],
                                  B["ab"], B["semD"]).wait()
            pltpu.make_async_copy(hl2d_hbm.at[pl.ds(base_r, BLK), :],
                                  B["hlb"], B["semD"]).wait()

        def compute(B):
            for k in range(BLK):
                for j in range(CH // 16):
                    col = k * CH + j * 16
                    e, r = _softmax4(B["ab"], col // 16)
                    B["sid"][k, pl.ds(j * 16, 16)] = B["hlb"][k, pl.ds(j * 16, 16)]
                    for f in range(NF):
                        B["tpb"][f * BLK + k, pl.ds(j * 16, 16)] = e[f] * r

        def fire_scat(B):
            for f in range(NF):
                for k in range(BLK):
                    pltpu.async_copy(B["tpb"].at[f * BLK + k],
                                     accsh[f].at[B["sid"].at[k]], B["semS"],
                                     add=True)

        def drain_scat(B):
            for f in range(NF):
                for k in range(BLK):
                    pltpu.make_async_copy(B["tpb"].at[f * BLK + k],
                                          accsh[f].at[B["sid"].at[k]],
                                          B["semS"]).wait()

        def zbody(i, _):
            lslice[pl.ds(i * 16, 16)] = zero
            return 0

        lax.fori_loop(0, TPN // 16, zbody, 0)
        base_row = s * TPN
        for f in range(NF):
            pltpu.sync_copy(lslice, accsh[f].at[pl.ds(base_row, TPN)])
        dummy = jnp.full((16,), NU, jnp.int32)
        for k in range(BLK):
            for j in range(CH // 16):
                bufs[1]["sid"][k, pl.ds(j * 16, 16)] = dummy
                for f in range(NF):
                    bufs[1]["tpb"][f * BLK + k, pl.ds(j * 16, 16)] = zero
        plsc.subcore_barrier()

        fire_dense(0, bufs[0])
        fire_dense(1, bufs[1])
        fire_scat(bufs[1])

        def body(i2, _):
            for b in (0, 1):
                blk = 2 * i2 + b
                ob = 1 - b
                drain_dense(bufs[b])
                compute(bufs[b])
                fire_scat(bufs[b])
                fire_dense(jnp.minimum(blk + 2, NBLK - 1), bufs[b])
                drain_scat(bufs[ob])
            return 0

        lax.fori_loop(0, NBLK // 2, body, 0)
        drain_scat(bufs[1])
        drain_dense(bufs[0])
        drain_dense(bufs[1])
        plsc.subcore_barrier()

        for f in range(NF):
            pltpu.sync_copy(accsh[f].at[pl.ds(base_row, TPN)], lslice)

            def finb(i, _):
                x = jnp.maximum(lslice[pl.ds(i * 16, 16)], 1e-8)
                lslice[pl.ds(i * 16, 16)] = _newton_rsqrt(x)
                return 0

            lax.fori_loop(0, TPN // 16, finb, 0)
            pltpu.sync_copy(lslice,
                            d_out[f].at[pl.ds(c * NPH + base_row, TPN)])

    return dsum


BLK = 3
BLKC = BLK * CH
NBLK = EPW // BLKC
NROWS = E2 // CH


def _k2_scratch():
    per_buf = [
        pltpu.VMEM((NF, BLKC), jnp.float32),
        pltpu.VMEM((BLK, CH), jnp.int32),
        pltpu.VMEM((BLK, CH), jnp.int32),
        pltpu.VMEM((BLK, CH), jnp.int32),
        pltpu.VMEM((BLK, CH), jnp.int32),
        pltpu.VMEM((BLK, CH), jnp.float32),
        pltpu.VMEM((BLK, CH), jnp.float32),
        pltpu.VMEM((BLKC, DF), jnp.float32),
        pltpu.VMEM((BLKC, DF), jnp.float32),
        pltpu.SemaphoreType.DMA,
        pltpu.SemaphoreType.DMA,
        pltpu.SemaphoreType.DMA,
    ]
    return per_buf + per_buf + [pltpu.VMEM_SHARED((NPH, DF), jnp.float32)]


@functools.cache
def _make_message(f: int):
    @functools.partial(
        pl.kernel,
        mesh=_mesh(),
        compiler_params=_SC_PARAMS,
        out_type=jax.ShapeDtypeStruct((NP, DF), jnp.float32),
        scratch_types=_k2_scratch(),
    )
    def message(a_hbm, d_hbm, t_dense_hbm, hl2d_hbm, tpg2d_hbm, f_hbm,
                *scr):
        bufs = [dict(zip(
            ("ab", "hlb", "tpb", "hpg", "sid", "dh", "dt", "trow", "vrow",
             "semD", "semG", "semS"), scr[12 * b:12 * (b + 1)]))
            for b in range(2)]
        facc = scr[24]
        c = lax.axis_index("c")
        s = lax.axis_index("s")
        base_e = c * EH + s * EPW
        base_r = (c * (EH // CH) + s * (EPW // CH))

        def dense_pairs(blk, B):
            gbase = base_e + blk * BLKC
            rowb = base_r + blk * BLK
            return [
                (a_hbm.at[:, pl.ds(gbase, BLKC)], B["ab"]),
                (hl2d_hbm.at[pl.ds(rowb, BLK), :], B["hlb"]),
                (tpg2d_hbm.at[pl.ds(rowb, BLK), :], B["tpb"]),
                (t_dense_hbm.at[pl.ds(gbase, BLKC),
                                pl.ds(f * DF, DF)], B["trow"]),
            ]

        def fire_dense(blk, B):
            for src, dst in dense_pairs(blk, B):
                pltpu.async_copy(src, dst, B["semD"])

        def drain_dense(B):
            for src, dst in dense_pairs(0, B):
                pltpu.make_async_copy(src, dst, B["semD"]).wait()

        def hpgsid(B):
            for k in range(BLK):
                for j in range(CH // 16):
                    hv = B["hlb"][k, pl.ds(j * 16, 16)]
                    B["hpg"][k, pl.ds(j * 16, 16)] = hv + c * NPH
                    B["sid"][k, pl.ds(j * 16, 16)] = hv

        def fire_gathers(B):
            for k in range(BLK):
                pltpu.async_copy(d_hbm.at[B["hpg"].at[k]], B["dh"].at[k],
                                 B["semG"])
                pltpu.async_copy(d_hbm.at[B["tpb"].at[k]], B["dt"].at[k],
                                 B["semG"])

        def drain_gathers(B):
            for k in range(BLK):
                pltpu.make_async_copy(d_hbm.at[B["hpg"].at[k]],
                                      B["dh"].at[k], B["semG"]).wait()
                pltpu.make_async_copy(d_hbm.at[B["tpb"].at[k]],
                                      B["dt"].at[k], B["semG"]).wait()

        def fire_scatters(B):
            for k in range(BLK):
                pltpu.async_copy(B["vrow"].at[pl.ds(k * CH, CH)],
                                 facc.at[B["sid"].at[k]], B["semS"],
                                 add=True)

        def drain_scatters(B):
            for k in range(BLK):
                pltpu.make_async_copy(B["vrow"].at[pl.ds(k * CH, CH)],
                                      facc.at[B["sid"].at[k]],
                                      B["semS"]).wait()

        def compute(B):
            ab, trow, vrow = B["ab"], B["trow"], B["vrow"]
            for k in range(BLK):
                for j in range(CH // 16):
                    col = k * CH + j * 16
                    a = [ab[ff, pl.ds(col, 16)] for ff in range(NF)]
                    m = jnp.maximum(jnp.maximum(a[0], a[1]),
                                    jnp.maximum(a[2], a[3]))
                    e = [jnp.exp(v - m) for v in a]
                    tp = e[f] / (e[0] + e[1] + e[2] + e[3])
                    dh = B["dh"][k, pl.ds(j * 16, 16)]
                    dt = B["dt"][k, pl.ds(j * 16, 16)]
                    ew = tp * dh * dt
                    for j2 in range(16):
                        row = col + j2
                        w = ew[j2]
                        vrow[row, pl.ds(0, 16)] = trow[row, pl.ds(0, 16)] * w
                        vrow[row, pl.ds(16, 16)] = trow[row, pl.ds(16, 16)] * w

        zero = jnp.zeros((16,), jnp.float32)
        dummy = jnp.full((16,), NU, jnp.int32)
        for i in range(BLKC):
            bufs[0]["vrow"][i, pl.ds(0, 16)] = zero
            bufs[0]["vrow"][i, pl.ds(16, 16)] = zero
            bufs[1]["vrow"][i, pl.ds(0, 16)] = zero
            bufs[1]["vrow"][i, pl.ds(16, 16)] = zero
        for k in range(BLK):
            for j in range(CH // 16):
                bufs[1]["sid"][k, pl.ds(j * 16, 16)] = dummy
        base_row = s * TPN
        for b in range(TPN // BLKC):
            pltpu.sync_copy(bufs[0]["vrow"],
                            facc.at[pl.ds(base_row + b * BLKC, BLKC)])
        rem = TPN - (TPN // BLKC) * BLKC
        pltpu.sync_copy(bufs[0]["vrow"].at[pl.ds(0, rem)],
                        facc.at[pl.ds(base_row + (TPN // BLKC) * BLKC, rem)])
        plsc.subcore_barrier()

        fire_dense(0, bufs[0])
        fire_dense(1, bufs[1])
        fire_scatters(bufs[1])
        drain_dense(bufs[0])
        hpgsid(bufs[0])
        fire_gathers(bufs[0])

        def body(i2, _):
            for b in (0, 1):
                blk = 2 * i2 + b
                ob = 1 - b
                drain_gathers(bufs[b])
                compute(bufs[b])
                fire_scatters(bufs[b])
                fire_dense(jnp.minimum(blk + 2, NBLK - 1), bufs[b])
                drain_scatters(bufs[ob])
                drain_dense(bufs[ob])
                hpgsid(bufs[ob])
                fire_gathers(bufs[ob])
            return 0

        lax.fori_loop(0, NBLK // 2, body, 0)

        drain_gathers(bufs[0])
        drain_scatters(bufs[1])
        drain_dense(bufs[1])
        plsc.subcore_barrier()
        pltpu.sync_copy(facc.at[pl.ds(base_row, TPN)],
                        f_hbm.at[pl.ds(c * NPH + base_row, TPN)])

    return message


GBLK = 2
GBLKC = GBLK * CH
GNBLK = EPW // GBLKC


@functools.cache
def _make_gather4():
    scratch = [
        pltpu.VMEM((GBLK, CH), jnp.int32),
        pltpu.VMEM((GBLK, CH), jnp.int32),
        pltpu.VMEM((GBLKC, EMB), jnp.float32),
        pltpu.VMEM((GBLKC, EMB), jnp.float32),
        pltpu.VMEM((GBLKC, EMB), jnp.float32),
        pltpu.SemaphoreType.DMA,
        pltpu.SemaphoreType.DMA,
        pltpu.SemaphoreType.DMA,
        pltpu.SemaphoreType.DMA,
        pltpu.SemaphoreType.DMA,
        pltpu.SemaphoreType.DMA,
        pltpu.SemaphoreType.DMA,
        pltpu.SemaphoreType.DMA,
    ]

    @functools.partial(
        pl.kernel,
        mesh=_mesh(),
        compiler_params=_SC_PARAMS,
        out_type=jax.ShapeDtypeStruct((E2, EMB), jnp.float32),
        scratch_types=scratch,
    )
    def gather4(idx2d_hbm, tab_hbm, out_hbm, *scr):
        ib = list(scr[0:2])
        rb = list(scr[2:5])
        semI = list(scr[5:7])
        semG = list(scr[7:10])
        semW = list(scr[10:13])
        c = lax.axis_index("c")
        s = lax.axis_index("s")
        base_e = c * EH + s * EPW
        base_r = c * (EH // CH) + s * (EPW // CH)

        def fire_idx(blk, i):
            pltpu.async_copy(
                idx2d_hbm.at[pl.ds(base_r + blk * GBLK, GBLK), :],
                ib[i], semI[i])

        def drain_idx(i):
            pltpu.make_async_copy(idx2d_hbm.at[pl.ds(base_r, GBLK), :],
                                  ib[i], semI[i]).wait()

        def fire_g(i, r):
            for k in range(GBLK):
                pltpu.async_copy(tab_hbm.at[ib[i].at[k]],
                                 rb[r].at[pl.ds(k * CH, CH)], semG[r])

        def drain_g(i, r):
            for k in range(GBLK):
                pltpu.make_async_copy(tab_hbm.at[ib[i].at[k]],
                                      rb[r].at[pl.ds(k * CH, CH)],
                                      semG[r]).wait()

        def fire_w(blk, r):
            pltpu.async_copy(rb[r],
                             out_hbm.at[pl.ds(base_e + blk * GBLKC, GBLKC), :],
                             semW[r])

        def drain_w(r):
            pltpu.make_async_copy(rb[r],
                                  out_hbm.at[pl.ds(base_e, GBLKC), :],
                                  semW[r]).wait()

        fire_idx(0, 0)
        fire_idx(1, 1)
        drain_idx(0)
        fire_g(0, 0)
        fire_w(1, 1)
        fire_w(2, 2)

        def section(sec, k):
            ibi, ibn = k % 2, (k + 1) % 2
            rbi, rbn = k % 3, (k + 1) % 3
            drain_w(rbn)
            drain_idx(ibn)
            fire_g(ibn, rbn)
            drain_g(ibi, rbi)
            fire_w(sec, rbi)
            fire_idx(jnp.minimum(sec + 2, GNBLK - 1), ibi)

        def body(i2, _):
            for k in range(6):
                section(6 * i2 + k, k)
            return 0

        nfull = GNBLK // 6
        lax.fori_loop(0, nfull, body, 0)
        for k in range(GNBLK - 6 * nfull):
            section(6 * nfull + k, k)
        last = GNBLK - 1
        drain_g((last + 1) % 2, (last + 1) % 3)
        drain_w((last - 1) % 3)
        drain_w(last % 3)
        drain_idx(last % 2)

    return gather4


_BN2 = 1024


def _pack_body(i0, i1, i2c, i3, o_ref):
    ins = (i0, i1, i2c, i3)
    for f in range(NF):
        o_ref[:, f * DF:(f + 1) * DF] = ins[f][...]


@functools.cache
def _make_pack4():
    return pl.pallas_call(
        _pack_body,
        grid=(NP // _BN2,),
        in_specs=[pl.BlockSpec((_BN2, DF), lambda i: (i, 0))
                  for _ in range(NF)],
        out_specs=pl.BlockSpec((_BN2, EMB), lambda i: (i, 0)),
        out_shape=jax.ShapeDtypeStruct((NP, EMB), jnp.float32),
    )


_BE = 4096


def _att_body(a_ref, g_ref, t_ref, o_ref):
    G = g_ref[...]
    T = t_ref[...]
    M = (lax.broadcasted_iota(jnp.int32, (EMB, NF), 0) // DF
         == lax.broadcasted_iota(jnp.int32, (EMB, NF), 1)
         ).astype(jnp.float32)
    Mt = (lax.broadcasted_iota(jnp.int32, (NF, EMB), 1) // DF
          == lax.broadcasted_iota(jnp.int32, (NF, EMB), 0)
          ).astype(jnp.float32)

    def mm(x, y):
        return jax.lax.dot(x, y, preferred_element_type=jnp.float32)

    ginv = 1.0 / jnp.maximum(jnp.sqrt(mm(G * G, M)), 1e-12)
    tinv = 1.0 / jnp.maximum(jnp.sqrt(mm(T * T, M)), 1e-12)
    prod = (G * mm(ginv, Mt)) * jnp.tanh(T * mm(tinv, Mt))
    u4 = mm(prod, M)
    o_ref[...] = a_ref[...] + u4.T


@functools.cache
def _make_att():
    return pl.pallas_call(
        _att_body,
        grid=(E2 // _BE,),
        in_specs=[
            pl.BlockSpec((NF, _BE), lambda i: (0, i)),
            pl.BlockSpec((_BE, EMB), lambda i: (i, 0)),
            pl.BlockSpec((_BE, EMB), lambda i: (i, 0)),
        ],
        out_specs=pl.BlockSpec((NF, _BE), lambda i: (0, i)),
        out_shape=jax.ShapeDtypeStruct((NF, E2), jnp.float32),
    )


_BN = 1024


def _mean_body(a_ref, b_ref, c_ref, o_ref):
    o_ref[...] = (a_ref[...] + b_ref[...] + c_ref[...]) * (1.0 / 3.0)


@functools.cache
def _make_mean():
    return pl.pallas_call(
        _mean_body,
        grid=(NP // _BN,),
        in_specs=[pl.BlockSpec((_BN, EMB), lambda i: (i, 0))
                  for _ in range(3)],
        out_specs=pl.BlockSpec((_BN, EMB), lambda i: (i, 0)),
        out_shape=jax.ShapeDtypeStruct((NP, EMB), jnp.float32),
    )


def kernel(user_emb, item_emb, all_h_list, all_t_list):
    h = all_h_list
    t = all_t_list
    hpg = h + jnp.where(h >= NU, NPH - NU, 0).astype(jnp.int32)
    tpg = t + jnp.where(t >= NU, NPH - NU, 0).astype(jnp.int32)
    padn = EH - EH_RAW
    hpg2 = jnp.concatenate([
        hpg[:EH_RAW], jnp.full((padn,), NU, jnp.int32),
        hpg[EH_RAW:], jnp.full((padn,), NPH + NU, jnp.int32),
    ])
    tpg2 = jnp.concatenate([
        tpg[:EH_RAW], jnp.zeros((padn,), jnp.int32),
        tpg[EH_RAW:], jnp.zeros((padn,), jnp.int32),
    ])
    hl2 = jnp.concatenate([hpg2[:EH], hpg2[EH:] - NPH])
    hl2d = hl2.reshape(E2 // CH, CH)
    tpg2d = tpg2.reshape(E2 // CH, CH)
    hpg2d = hpg2.reshape(E2 // CH, CH)

    zpad = jnp.zeros((NPH - NU, EMB), jnp.float32)
    ego = jnp.concatenate([user_emb, zpad, item_emb, zpad], axis=0)

    dsum = _make_dsum()
    gather4 = _make_gather4()
    att = _make_att()
    pack4 = _make_pack4()
    msg = [_make_message(f) for f in range(NF)]

    a_val = jnp.ones((NF, E2), jnp.float32)
    all_layers = [ego]
    for layer in range(2):
        tails = gather4(tpg2d, ego)
        fpack = None
        for it in range(2):
            d = dsum(a_val, hl2d)
            layer_f = [msg[f](a_val, d[f], tails, hl2d, tpg2d)
                       for f in range(NF)]
            fpack = pack4(*layer_f)
            last = layer == 1 and it == 1
            if not last:
                heads = gather4(hpg2d, fpack)
                a_val = att(a_val, heads, tails)
        ego = fpack
        all_layers.append(ego)

    mean = _make_mean()
    emb = mean(*all_layers)
    u_g = emb[:NU, :]
    i_g = emb[NPH:NPH + NI, :]
    return (u_g, i_g)
